# Initial kernel scaffold; baseline (speedup 1.0000x reference)
import jax
import jax.numpy as jnp
from jax.experimental import pallas as pl


def kernel(x, edge_index, edge_attr, batch, assignment_index_3, iso_type_3, edge_index_3, batch_3, W1a, b1a, W1b, b1b, root1, bias1, W2a, b2a, W2b, b2b, root2, bias2, W3a, b3a, W3b, b3b, root3, bias3, Wrel6, brel6, Wroot6, Wrel7, brel7, Wroot7, fc1_W, fc1_b, fc2_W, fc2_b, fc3_W, fc3_b):
    raise NotImplementedError("write your pallas kernel here")


# speedup vs baseline: 2.3284x; 2.3284x over previous
import functools

import jax
import jax.numpy as jnp
from jax import lax
from jax.experimental import pallas as pl
from jax.experimental.pallas import tpu as pltpu
from jax.experimental.pallas import tpu_sc as plsc

F32 = jnp.float32

_N = 12000
_E = 24000
_B = 1024
_N3 = 30000
_A = 90000
_E3 = 120000
_F_IN = 13
_NI3 = 16

NC, NS = 2, 16
NW = NC * NS
CK = 128

NPAD = 12288
EPAD = 24576
N3PAD = 30720
APAD = 90112
E3PAD = 122880
BPAD = 1280

DUM_N = NPAD - 8
DUM_N3 = N3PAD - 8
DUM_B = BPAD - 8

_MESH = plsc.VectorSubcoreMesh(
    core_axis_name="c", subcore_axis_name="s", num_cores=NC, num_subcores=NS)
_SC_PARAMS = pltpu.CompilerParams(use_tc_tiling_on_sc=False)


def _make_gather(d, nchunks):
    ept = nchunks * CK

    @functools.partial(
        pl.kernel,
        out_type=jax.ShapeDtypeStruct((NW * ept, d), F32),
        mesh=_MESH,
        compiler_params=_SC_PARAMS,
        scratch_types=[
            pltpu.VMEM((nchunks, CK), jnp.int32),
            pltpu.VMEM((ept, d), F32),
        ],
    )
    def k(table, idx, out, idx_v, buf):
        cid = lax.axis_index("c")
        sid = lax.axis_index("s")
        wid = sid * NC + cid
        pltpu.sync_copy(idx.at[wid], idx_v)

        def body(j, carry):
            pltpu.sync_copy(table.at[idx_v.at[j]], buf.at[pl.ds(j * CK, CK)])
            return carry

        lax.fori_loop(0, nchunks, body, 0)
        pltpu.sync_copy(buf, out.at[pl.ds(wid * ept, ept)])

    return k


def _make_scatter(d, nchunks, rows):
    ept = nchunks * CK
    rz = rows // NS

    @functools.partial(
        pl.kernel,
        out_type=jax.ShapeDtypeStruct((2 * rows, d), F32),
        mesh=_MESH,
        compiler_params=_SC_PARAMS,
        scratch_types=[
            pltpu.VMEM((nchunks, CK), jnp.int32),
            pltpu.VMEM((CK, d), F32),
            pltpu.VMEM_SHARED((rows, d), F32),
        ],
    )
    def k(data, idx, zeros, out, idx_v, buf, acc):
        cid = lax.axis_index("c")
        sid = lax.axis_index("s")
        wid = sid * NC + cid
        pltpu.sync_copy(zeros.at[pl.ds(sid * rz, rz)], acc.at[pl.ds(sid * rz, rz)])
        pltpu.sync_copy(idx.at[wid], idx_v)
        plsc.subcore_barrier()

        def body(j, carry):
            pltpu.sync_copy(data.at[pl.ds(wid * ept + j * CK, CK)], buf)
            pltpu.sync_copy(buf, acc.at[idx_v.at[j]], add=True)
            return carry

        lax.fori_loop(0, nchunks, body, 0)
        plsc.subcore_barrier()
        pltpu.sync_copy(acc.at[pl.ds(sid * rz, rz)],
                        out.at[pl.ds(cid * rows + sid * rz, rz)])

    return k


def _make_pool_split(nchunks, rows):
    rz = rows // NS

    @functools.partial(
        pl.kernel,
        out_type=jax.ShapeDtypeStruct((2 * rows, 32), F32),
        mesh=_MESH,
        compiler_params=_SC_PARAMS,
        scratch_types=[
            pltpu.VMEM((nchunks, CK), jnp.int32),
            pltpu.VMEM((nchunks, CK), jnp.int32),
            pltpu.VMEM((CK, 32), F32),
            pltpu.VMEM_SHARED((rows, 32), F32),
        ],
    )
    def k(ta, tb, rowi, coli, zeros, out, row_v, col_v, buf, acc):
        cid = lax.axis_index("c")
        sid = lax.axis_index("s")
        pltpu.sync_copy(zeros.at[pl.ds(sid * rz, rz)], acc.at[pl.ds(sid * rz, rz)])
        pltpu.sync_copy(rowi.at[sid], row_v)
        pltpu.sync_copy(coli.at[sid], col_v)
        plsc.subcore_barrier()

        def body(j, carry):
            @pl.when(cid == 0)
            def _():
                pltpu.sync_copy(ta.at[row_v.at[j]], buf)

            @pl.when(cid == 1)
            def _():
                pltpu.sync_copy(tb.at[row_v.at[j]], buf)

            pltpu.sync_copy(buf, acc.at[col_v.at[j]], add=True)
            return carry

        lax.fori_loop(0, nchunks, body, 0)
        plsc.subcore_barrier()
        pltpu.sync_copy(acc.at[pl.ds(sid * rz, rz)],
                        out.at[pl.ds(cid * rows + sid * rz, rz)])

    return k


def _make_pool_batch_split(nchunks, nt):
    ept = nchunks * CK
    rz = BPAD // NS

    @functools.partial(
        pl.kernel,
        out_type=(jax.ShapeDtypeStruct((2 * BPAD, 32), F32),
                  jax.ShapeDtypeStruct((BPAD, 16), F32)),
        mesh=_MESH,
        compiler_params=_SC_PARAMS,
        scratch_types=[
            pltpu.VMEM((nchunks, CK), jnp.int32),
            pltpu.VMEM((CK, 32), F32),
            pltpu.VMEM((CK, 16), F32),
            pltpu.VMEM_SHARED((BPAD, 32), F32),
            pltpu.VMEM_SHARED((BPAD, 16), F32),
        ],
    )
    def k(ta, tb, coli, zeros, zeros16, ones, outs, outc,
          col_v, buf, ones_v, acc, accc):
        cid = lax.axis_index("c")
        sid = lax.axis_index("s")
        pltpu.sync_copy(zeros.at[pl.ds(sid * rz, rz)], acc.at[pl.ds(sid * rz, rz)])
        pltpu.sync_copy(zeros16.at[pl.ds(sid * rz, rz)],
                        accc.at[pl.ds(sid * rz, rz)])
        pltpu.sync_copy(coli.at[sid], col_v)
        pltpu.sync_copy(ones, ones_v)
        plsc.subcore_barrier()

        def body(j, carry):
            @pl.when(cid == 0)
            def _():
                pltpu.sync_copy(ta.at[pl.ds(sid * ept + j * CK, CK)], buf)
                pltpu.sync_copy(ones_v, accc.at[col_v.at[j]], add=True)

            @pl.when(cid == 1)
            def _():
                pltpu.sync_copy(tb.at[pl.ds(sid * ept + j * CK, CK)], buf)

            pltpu.sync_copy(buf, acc.at[col_v.at[j]], add=True)
            return carry

        lax.fori_loop(0, nchunks, body, 0)
        plsc.subcore_barrier()
        pltpu.sync_copy(acc.at[pl.ds(sid * rz, rz)],
                        outs.at[pl.ds(cid * BPAD + sid * rz, rz)])

        @pl.when(cid == 0)
        def _():
            pltpu.sync_copy(accc.at[pl.ds(sid * rz, rz)],
                            outc.at[pl.ds(sid * rz, rz)])

    return k


def _make_counts(nchunks, rows):
    rz = rows // NS

    @functools.partial(
        pl.kernel,
        out_type=jax.ShapeDtypeStruct((2 * rows, 16), F32),
        mesh=_MESH,
        compiler_params=_SC_PARAMS,
        scratch_types=[
            pltpu.VMEM((nchunks, CK), jnp.int32),
            pltpu.VMEM((CK, 16), F32),
            pltpu.VMEM_SHARED((rows, 16), F32),
        ],
    )
    def k(coli, zeros16, ones, out, col_v, ones_v, acc):
        cid = lax.axis_index("c")
        sid = lax.axis_index("s")
        wid = sid * NC + cid
        pltpu.sync_copy(zeros16.at[pl.ds(sid * rz, rz)], acc.at[pl.ds(sid * rz, rz)])
        pltpu.sync_copy(coli.at[wid], col_v)
        pltpu.sync_copy(ones, ones_v)
        plsc.subcore_barrier()

        def body(j, carry):
            pltpu.sync_copy(ones_v, acc.at[col_v.at[j]], add=True)
            return carry

        lax.fori_loop(0, nchunks, body, 0)
        plsc.subcore_barrier()
        pltpu.sync_copy(acc.at[pl.ds(sid * rz, rz)],
                        out.at[pl.ds(cid * rows + sid * rz, rz)])

    return k


def _elu(a):
    return jnp.where(a > 0, a, jnp.exp(jnp.minimum(a, 0.0)) - 1.0)


def _make_msg(m_in_pad, m_out, be=256):
    kin = m_in_pad * 128

    def body(xg, ea, wa, ba, wb2, bb2, out):
        h = jnp.maximum(ea[...] @ wa[...] + ba[...], 0.0)
        xgv = xg[...]
        v = (xgv[:, :, None] * h[:, None, :]).reshape(be, kin)
        out[...] = (
            lax.dot_general(v, wb2[...], (((1,), (0,)), ((), ())),
                            preferred_element_type=F32)
            + xgv @ bb2[...])

    return pl.pallas_call(
        body,
        grid=(EPAD // be,),
        in_specs=[
            pl.BlockSpec((be, m_in_pad), lambda i: (i, 0)),
            pl.BlockSpec((be, 8), lambda i: (i, 0)),
            pl.BlockSpec((8, 128), lambda i: (0, 0)),
            pl.BlockSpec((1, 128), lambda i: (0, 0)),
            pl.BlockSpec((kin, m_out), lambda i: (0, 0)),
            pl.BlockSpec((m_in_pad, m_out), lambda i: (0, 0)),
        ],
        out_specs=pl.BlockSpec((be, m_out), lambda i: (i, 0)),
        out_shape=jax.ShapeDtypeStruct((EPAD, m_out), F32),
    )


def _make_node(m_in_pad, d, npad, split=False, bn=512):
    nb = npad // bn

    def body(p0, p1, xb, root, bias, *outs):
        a = _elu(p0[...] + p1[...] + xb[...] @ root[...] + bias[...])
        if split:
            outs[0][...] = a[:, :32]
            outs[1][...] = a[:, 32:]
        else:
            outs[0][...] = a

    if split:
        out_specs = (pl.BlockSpec((bn, 32), lambda i: (i, 0)),
                     pl.BlockSpec((bn, 32), lambda i: (i, 0)))
        out_shape = (jax.ShapeDtypeStruct((npad, 32), F32),
                     jax.ShapeDtypeStruct((npad, 32), F32))
    else:
        out_specs = pl.BlockSpec((bn, d), lambda i: (i, 0))
        out_shape = jax.ShapeDtypeStruct((npad, d), F32)

    return pl.pallas_call(
        body,
        grid=(nb,),
        in_specs=[
            pl.BlockSpec((bn, d), lambda i: (i, 0)),
            pl.BlockSpec((bn, d), lambda i: (i + nb, 0)),
            pl.BlockSpec((bn, m_in_pad), lambda i: (i, 0)),
            pl.BlockSpec((m_in_pad, d), lambda i: (0, 0)),
            pl.BlockSpec((1, d), lambda i: (0, 0)),
        ],
        out_specs=out_specs,
        out_shape=out_shape,
    )


def _make_gc_pre(bn=512):
    nb = N3PAD // bn

    def body(s_lo, s_hi, c0, c1, iso, wrel_a, wrel_b, wroot_a, wroot_b, brel,
             ta_out, tb_out, r_out):
        cnt = jnp.maximum(c0[...] + c1[...], 1.0)[:, 0:1]
        h3m = jnp.concatenate([s_lo[...], s_hi[...]], axis=1) / cnt
        isov = iso[...]
        t = h3m @ wrel_a[...] + isov @ wrel_b[...]
        ta_out[...] = t[:, :32]
        tb_out[...] = t[:, 32:]
        r_out[...] = h3m @ wroot_a[...] + isov @ wroot_b[...] + brel[...]

    return pl.pallas_call(
        body,
        grid=(nb,),
        in_specs=[
            pl.BlockSpec((bn, 32), lambda i: (i, 0)),
            pl.BlockSpec((bn, 32), lambda i: (i + nb, 0)),
            pl.BlockSpec((bn, 16), lambda i: (i, 0)),
            pl.BlockSpec((bn, 16), lambda i: (i + nb, 0)),
            pl.BlockSpec((bn, 16), lambda i: (i, 0)),
            pl.BlockSpec((64, 64), lambda i: (0, 0)),
            pl.BlockSpec((16, 64), lambda i: (0, 0)),
            pl.BlockSpec((64, 64), lambda i: (0, 0)),
            pl.BlockSpec((16, 64), lambda i: (0, 0)),
            pl.BlockSpec((1, 64), lambda i: (0, 0)),
        ],
        out_specs=(pl.BlockSpec((bn, 32), lambda i: (i, 0)),
                   pl.BlockSpec((bn, 32), lambda i: (i, 0)),
                   pl.BlockSpec((bn, 64), lambda i: (i, 0))),
        out_shape=(jax.ShapeDtypeStruct((N3PAD, 32), F32),
                   jax.ShapeDtypeStruct((N3PAD, 32), F32),
                   jax.ShapeDtypeStruct((N3PAD, 64), F32)),
    )


def _make_gc_mid(bn=512):
    nb = N3PAD // bn

    def body(a_lo, a_hi, r6, wrel, wroot, brel, ta_out, tb_out, r_out):
        h3b = _elu(jnp.concatenate([a_lo[...], a_hi[...]], axis=1) + r6[...])
        t = h3b @ wrel[...]
        ta_out[...] = t[:, :32]
        tb_out[...] = t[:, 32:]
        r_out[...] = h3b @ wroot[...] + brel[...]

    return pl.pallas_call(
        body,
        grid=(nb,),
        in_specs=[
            pl.BlockSpec((bn, 32), lambda i: (i, 0)),
            pl.BlockSpec((bn, 32), lambda i: (i + nb, 0)),
            pl.BlockSpec((bn, 64), lambda i: (i, 0)),
            pl.BlockSpec((64, 64), lambda i: (0, 0)),
            pl.BlockSpec((64, 64), lambda i: (0, 0)),
            pl.BlockSpec((1, 64), lambda i: (0, 0)),
        ],
        out_specs=(pl.BlockSpec((bn, 32), lambda i: (i, 0)),
                   pl.BlockSpec((bn, 32), lambda i: (i, 0)),
                   pl.BlockSpec((bn, 64), lambda i: (i, 0))),
        out_shape=(jax.ShapeDtypeStruct((N3PAD, 32), F32),
                   jax.ShapeDtypeStruct((N3PAD, 32), F32),
                   jax.ShapeDtypeStruct((N3PAD, 64), F32)),
    )


def _make_gc_post(bn=512):
    nb = N3PAD // bn

    def body(a_lo, a_hi, r7, fa_out, fb_out):
        a = _elu(jnp.concatenate([a_lo[...], a_hi[...]], axis=1) + r7[...])
        fa_out[...] = a[:, :32]
        fb_out[...] = a[:, 32:]

    return pl.pallas_call(
        body,
        grid=(nb,),
        in_specs=[
            pl.BlockSpec((bn, 32), lambda i: (i, 0)),
            pl.BlockSpec((bn, 32), lambda i: (i + nb, 0)),
            pl.BlockSpec((bn, 64), lambda i: (i, 0)),
        ],
        out_specs=(pl.BlockSpec((bn, 32), lambda i: (i, 0)),
                   pl.BlockSpec((bn, 32), lambda i: (i, 0))),
        out_shape=(jax.ShapeDtypeStruct((N3PAD, 32), F32),
                   jax.ShapeDtypeStruct((N3PAD, 32), F32)),
    )


def _make_head():
    def body(s10, s11, c1, s30, s31, c3,
             w1a, w1b, b1, w2, b2, w3r, b3, out):
        cnt1 = jnp.maximum(c1[...], 1.0)[:, 0:1]
        x1 = jnp.concatenate([s10[...], s11[...]], axis=1) / cnt1
        cnt3 = jnp.maximum(c3[...], 1.0)[:, 0:1]
        x3 = jnp.concatenate([s30[...], s31[...]], axis=1) / cnt3
        y = _elu(x1 @ w1a[...] + x3 @ w1b[...] + b1[...])
        y = _elu(y @ w2[...] + b2[...])
        out[...] = jnp.sum(y * w3r[...], axis=1, keepdims=True) + b3[...]

    bs = lambda shape: pl.BlockSpec(shape, lambda i: (0, 0))
    return pl.pallas_call(
        body,
        grid=(1,),
        in_specs=[
            pl.BlockSpec((BPAD, 32), lambda i: (0, 0)),
            pl.BlockSpec((BPAD, 32), lambda i: (1, 0)),
            bs((BPAD, 16)),
            pl.BlockSpec((BPAD, 32), lambda i: (0, 0)),
            pl.BlockSpec((BPAD, 32), lambda i: (1, 0)),
            bs((BPAD, 16)),
            bs((64, 64)), bs((64, 64)), bs((1, 64)),
            bs((64, 32)), bs((1, 32)), bs((1, 32)), bs((1, 1)),
        ],
        out_specs=pl.BlockSpec((BPAD, 1), lambda i: (0, 0)),
        out_shape=jax.ShapeDtypeStruct((BPAD, 1), F32),
    )


def _pad_idx(idx, n_pad, fill):
    v = jnp.full((n_pad,), fill, jnp.int32)
    return v.at[: idx.shape[0]].set(idx.astype(jnp.int32))


def _prep_nnconv(Wa, ba, Wb, bb, root, bias, m_in, m_in_pad, m_out):
    wa8 = jnp.zeros((8, 128), F32).at[:6].set(Wa)
    ba2 = ba.reshape(1, 128)
    wb3 = Wb.reshape(128, m_in, m_out).transpose(1, 0, 2)
    wb2 = jnp.zeros((m_in_pad, 128, m_out), F32).at[:m_in].set(wb3)
    wb2 = wb2.reshape(m_in_pad * 128, m_out)
    bb2 = jnp.zeros((m_in_pad, m_out), F32).at[:m_in].set(bb.reshape(m_in, m_out))
    rootp = jnp.zeros((m_in_pad, m_out), F32).at[:m_in].set(root)
    bias2 = bias.reshape(1, m_out)
    return wa8, ba2, wb2, bb2, rootp, bias2


def kernel(x, edge_index, edge_attr, batch, assignment_index_3, iso_type_3,
           edge_index_3, batch_3, W1a, b1a, W1b, b1b, root1, bias1,
           W2a, b2a, W2b, b2b, root2, bias2, W3a, b3a, W3b, b3b, root3, bias3,
           Wrel6, brel6, Wroot6, Wrel7, brel7, Wroot7,
           fc1_W, fc1_b, fc2_W, fc2_b, fc3_W, fc3_b):
    xpad = jnp.zeros((NPAD, 16), F32).at[:_N, :_F_IN].set(x)
    eapad = jnp.zeros((EPAD, 8), F32).at[:_E, :6].set(edge_attr)
    src_i = _pad_idx(edge_index[0], EPAD, 0).reshape(NW, 6, CK)
    dst_i = _pad_idx(edge_index[1], EPAD, DUM_N).reshape(NW, 6, CK)
    row3_i = _pad_idx(assignment_index_3[0], APAD, 0).reshape(NS, 44, CK)
    col3_i = _pad_idx(assignment_index_3[1], APAD, DUM_N3).reshape(NS, 44, CK)
    col3_w = _pad_idx(assignment_index_3[1], APAD, DUM_N3).reshape(NW, 22, CK)
    src3_i = _pad_idx(edge_index_3[0], E3PAD, 0).reshape(NS, 60, CK)
    dst3_i = _pad_idx(edge_index_3[1], E3PAD, DUM_N3).reshape(NS, 60, CK)
    batch_i = _pad_idx(batch, NPAD, DUM_B).reshape(NS, 6, CK)
    batch3_i = _pad_idx(batch_3, N3PAD, DUM_B).reshape(NS, 15, CK)
    isopad = jnp.zeros((N3PAD, 16), F32).at[:_N3].set(iso_type_3)

    zN32 = jnp.zeros((NPAD, 32), F32)
    zN64 = jnp.zeros((NPAD, 64), F32)
    z32N3 = jnp.zeros((N3PAD, 32), F32)
    z16N3 = jnp.zeros((N3PAD, 16), F32)
    z32B = jnp.zeros((BPAD, 32), F32)
    z16B = jnp.zeros((BPAD, 16), F32)
    ones128 = jnp.ones((CK, 16), F32)

    p1w = _prep_nnconv(W1a, b1a, W1b, b1b, root1, bias1, _F_IN, 16, 32)
    p2w = _prep_nnconv(W2a, b2a, W2b, b2b, root2, bias2, 32, 32, 64)
    p3w = _prep_nnconv(W3a, b3a, W3b, b3b, root3, bias3, 64, 64, 64)

    xg1 = _make_gather(16, 6)(xpad, src_i)
    msg1 = _make_msg(16, 32)(xg1, eapad, *p1w[:4])
    agg1 = _make_scatter(32, 6, NPAD)(msg1, dst_i, zN32)
    h1 = _make_node(16, 32, NPAD)(agg1, agg1, xpad, p1w[4], p1w[5])

    xg2 = _make_gather(32, 6)(h1, src_i)
    msg2 = _make_msg(32, 64)(xg2, eapad, *p2w[:4])
    agg2 = _make_scatter(64, 6, NPAD)(msg2, dst_i, zN64)
    h2 = _make_node(32, 64, NPAD)(agg2, agg2, h1, p2w[4], p2w[5])

    xg3 = _make_gather(64, 6)(h2, src_i)
    msg3 = _make_msg(64, 64)(xg3, eapad, *p3w[:4])
    agg3 = _make_scatter(64, 6, NPAD)(msg3, dst_i, zN64)
    ha, hb = _make_node(64, 64, NPAD, split=True)(agg3, agg3, h2, p3w[4], p3w[5])

    s3sum = _make_pool_split(44, N3PAD)(ha, hb, row3_i, col3_i, z32N3)
    ccol = _make_counts(22, N3PAD)(col3_w, z16N3, ones128)
    s1p, c1p = _make_pool_batch_split(6, NPAD)(
        ha, hb, batch_i, z32B, z16B, ones128)

    t6a, t6b, r6 = _make_gc_pre()(
        s3sum, s3sum, ccol, ccol, isopad,
        Wrel6[:64], Wrel6[64:], Wroot6[:64], Wroot6[64:], brel6.reshape(1, 64))
    agg6 = _make_pool_split(60, N3PAD)(t6a, t6b, src3_i, dst3_i, z32N3)
    t7a, t7b, r7 = _make_gc_mid()(
        agg6, agg6, r6, Wrel7, Wroot7, brel7.reshape(1, 64))
    agg7 = _make_pool_split(60, N3PAD)(t7a, t7b, src3_i, dst3_i, z32N3)
    fa, fb = _make_gc_post()(agg7, agg7, r7)
    s3p, c3p = _make_pool_batch_split(15, N3PAD)(
        fa, fb, batch3_i, z32B, z16B, ones128)

    out = _make_head()(
        s1p, s1p, c1p, s3p, s3p, c3p,
        fc1_W[:64], fc1_W[64:], fc1_b.reshape(1, 64),
        fc2_W, fc2_b.reshape(1, 32),
        fc3_W.reshape(1, 32), fc3_b.reshape(1, 1))
    return out[:_B, 0]


# speedup vs baseline: 2.4381x; 1.0471x over previous
import functools

import jax
import jax.numpy as jnp
from jax import lax
from jax.experimental import pallas as pl
from jax.experimental.pallas import tpu as pltpu
from jax.experimental.pallas import tpu_sc as plsc

F32 = jnp.float32

_N = 12000
_E = 24000
_B = 1024
_N3 = 30000
_A = 90000
_E3 = 120000
_F_IN = 13
_NI3 = 16

NC, NS = 2, 16
NW = NC * NS
CK = 128

NPAD = 12288
EPAD = 24576
N3PAD = 30720
APAD = 90112
E3PAD = 122880
BPAD = 1280

DUM_N = NPAD - 8
DUM_N3 = N3PAD - 8
DUM_B = BPAD - 8

_MESH = plsc.VectorSubcoreMesh(
    core_axis_name="c", subcore_axis_name="s", num_cores=NC, num_subcores=NS)
_SC_PARAMS = pltpu.CompilerParams(use_tc_tiling_on_sc=False)
_SDS = jax.ShapeDtypeStruct


def _make_gather(d, nchunks):
    ept = nchunks * CK

    @functools.partial(
        pl.kernel,
        out_type=_SDS((NW * ept, d), F32),
        mesh=_MESH,
        compiler_params=_SC_PARAMS,
        scratch_types=[
            pltpu.VMEM((nchunks, CK), jnp.int32),
            pltpu.VMEM((ept, d), F32),
            pltpu.SemaphoreType.DMA,
        ],
    )
    def k(table, idx, out, idx_v, buf, sem):
        cid = lax.axis_index("c")
        sid = lax.axis_index("s")
        wid = sid * NC + cid
        pltpu.sync_copy(idx.at[wid], idx_v)
        descs = [
            pltpu.async_copy(table.at[idx_v.at[j]],
                             buf.at[pl.ds(j * CK, CK)], sem)
            for j in range(nchunks)
        ]
        for dsc in descs:
            dsc.wait()
        pltpu.sync_copy(buf, out.at[pl.ds(wid * ept, ept)])

    return k


def _make_scatter(d, nchunks, rows, u=3):
    ept = nchunks * CK
    rz = rows // NS
    ng = nchunks // u

    @functools.partial(
        pl.kernel,
        out_type=_SDS((2 * rows, d), F32),
        mesh=_MESH,
        compiler_params=_SC_PARAMS,
        scratch_types=[
            pltpu.VMEM((nchunks, CK), jnp.int32),
            pltpu.VMEM((u * CK, d), F32),
            pltpu.VMEM_SHARED((rows, d), F32),
        ] + [pltpu.SemaphoreType.DMA] * (2 * u),
    )
    def k(data, idx, zeros, out, idx_v, bufs, acc, *sems):
        gs, ss = sems[:u], sems[u:]
        cid = lax.axis_index("c")
        sid = lax.axis_index("s")
        wid = sid * NC + cid
        pltpu.sync_copy(zeros.at[pl.ds(sid * rz, rz)], acc.at[pl.ds(sid * rz, rz)])
        pltpu.sync_copy(idx.at[wid], idx_v)
        plsc.subcore_barrier()

        def grp(g, carry):
            gd = []
            for b in range(u):
                j = g * u + b
                gd.append(pltpu.async_copy(
                    data.at[pl.ds(wid * ept + j * CK, CK)],
                    bufs.at[pl.ds(b * CK, CK)], gs[b]))
            sd = []
            for b in range(u):
                j = g * u + b
                gd[b].wait()
                sd.append(pltpu.async_copy(
                    bufs.at[pl.ds(b * CK, CK)], acc.at[idx_v.at[j]],
                    ss[b], add=True))
            for dsc in sd:
                dsc.wait()
            return carry

        lax.fori_loop(0, ng, grp, 0)
        plsc.subcore_barrier()
        pltpu.sync_copy(acc.at[pl.ds(sid * rz, rz)],
                        out.at[pl.ds(cid * rows + sid * rz, rz)])

    return k


def _fire_half_gather(cid, ta, tb, idx_row, dst, sem):
    @pl.when(cid == 0)
    def _():
        pltpu.async_copy(ta.at[idx_row], dst, sem)

    @pl.when(cid == 1)
    def _():
        pltpu.async_copy(tb.at[idx_row], dst, sem)


def _make_pool_split(nchunks, rows, u=4):
    rz = rows // NS
    ng = nchunks // u

    @functools.partial(
        pl.kernel,
        out_type=_SDS((2 * rows, 32), F32),
        mesh=_MESH,
        compiler_params=_SC_PARAMS,
        scratch_types=[
            pltpu.VMEM((nchunks, CK), jnp.int32),
            pltpu.VMEM((nchunks, CK), jnp.int32),
            pltpu.VMEM((u * CK, 32), F32),
            pltpu.VMEM_SHARED((rows, 32), F32),
        ] + [pltpu.SemaphoreType.DMA] * (2 * u),
    )
    def k(ta, tb, rowi, coli, zeros, out, row_v, col_v, bufs, acc, *sems):
        gs, ss = sems[:u], sems[u:]
        cid = lax.axis_index("c")
        sid = lax.axis_index("s")
        pltpu.sync_copy(zeros.at[pl.ds(sid * rz, rz)], acc.at[pl.ds(sid * rz, rz)])
        pltpu.sync_copy(rowi.at[sid], row_v)
        pltpu.sync_copy(coli.at[sid], col_v)
        plsc.subcore_barrier()

        def grp(g, carry):
            for b in range(u):
                j = g * u + b
                _fire_half_gather(cid, ta, tb, row_v.at[j],
                                  bufs.at[pl.ds(b * CK, CK)], gs[b])
            sd = []
            for b in range(u):
                j = g * u + b
                dst = bufs.at[pl.ds(b * CK, CK)]
                pltpu.make_async_copy(ta.at[row_v.at[j]], dst, gs[b]).wait()
                sd.append(pltpu.async_copy(dst, acc.at[col_v.at[j]],
                                           ss[b], add=True))
            for dsc in sd:
                dsc.wait()
            return carry

        lax.fori_loop(0, ng, grp, 0)
        plsc.subcore_barrier()
        pltpu.sync_copy(acc.at[pl.ds(sid * rz, rz)],
                        out.at[pl.ds(cid * rows + sid * rz, rz)])

    return k


def _make_pool3(u=4, ub=3):
    rzn = N3PAD // NS
    rzb = BPAD // NS
    nga = 44 // u
    ngb = 6 // ub

    @functools.partial(
        pl.kernel,
        out_type=(_SDS((2 * N3PAD, 32), F32), _SDS((N3PAD, 16), F32),
                  _SDS((2 * BPAD, 32), F32), _SDS((BPAD, 16), F32)),
        mesh=_MESH,
        compiler_params=_SC_PARAMS,
        scratch_types=[
            pltpu.VMEM((44, CK), jnp.int32),
            pltpu.VMEM((44, CK), jnp.int32),
            pltpu.VMEM((6, CK), jnp.int32),
            pltpu.VMEM((u * CK, 32), F32),
            pltpu.VMEM((CK, 16), F32),
            pltpu.VMEM_SHARED((N3PAD, 32), F32),
            pltpu.VMEM_SHARED((N3PAD, 16), F32),
            pltpu.VMEM_SHARED((BPAD, 32), F32),
            pltpu.VMEM_SHARED((BPAD, 16), F32),
        ] + [pltpu.SemaphoreType.DMA] * (3 * u),
    )
    def k(ta, tb, rowi, coli, bati, z32n, z16n, z32b, z16b, ones,
          outs, outc, outb, outbc,
          row_v, col_v, bat_v, bufs, ones_v, accs, accc, accb, accbc, *sems):
        gs, ss, cs = sems[:u], sems[u:2 * u], sems[2 * u:]
        cid = lax.axis_index("c")
        sid = lax.axis_index("s")
        pltpu.sync_copy(z32n.at[pl.ds(sid * rzn, rzn)],
                        accs.at[pl.ds(sid * rzn, rzn)])
        pltpu.sync_copy(z16n.at[pl.ds(sid * rzn, rzn)],
                        accc.at[pl.ds(sid * rzn, rzn)])
        pltpu.sync_copy(z32b.at[pl.ds(sid * rzb, rzb)],
                        accb.at[pl.ds(sid * rzb, rzb)])
        pltpu.sync_copy(z16b.at[pl.ds(sid * rzb, rzb)],
                        accbc.at[pl.ds(sid * rzb, rzb)])
        pltpu.sync_copy(rowi.at[sid], row_v)
        pltpu.sync_copy(coli.at[sid], col_v)
        pltpu.sync_copy(bati.at[sid], bat_v)
        pltpu.sync_copy(ones, ones_v)
        plsc.subcore_barrier()

        def grp_a(g, carry):
            for b in range(u):
                j = g * u + b
                _fire_half_gather(cid, ta, tb, row_v.at[j],
                                  bufs.at[pl.ds(b * CK, CK)], gs[b])
            sd = []
            for b in range(u):
                j = g * u + b
                dst = bufs.at[pl.ds(b * CK, CK)]
                pltpu.make_async_copy(ta.at[row_v.at[j]], dst, gs[b]).wait()
                sd.append(pltpu.async_copy(dst, accs.at[col_v.at[j]],
                                           ss[b], add=True))

                @pl.when(cid == 0)
                def _(j=j, b=b):
                    pltpu.async_copy(ones_v, accc.at[col_v.at[j]],
                                     cs[b], add=True)

            for b in range(u):
                sd[b].wait()
                j = g * u + b

                @pl.when(cid == 0)
                def _(j=j, b=b):
                    pltpu.make_async_copy(ones_v, accc.at[col_v.at[j]],
                                          cs[b]).wait()

            return carry

        lax.fori_loop(0, nga, grp_a, 0)

        def grp_b(g, carry):
            for b in range(ub):
                j = g * ub + b
                _fire_half_gather(cid, ta, tb,
                                  pl.ds(sid * 768 + j * CK, CK),
                                  bufs.at[pl.ds(b * CK, CK)], gs[b])
            sd = []
            for b in range(ub):
                j = g * ub + b
                dst = bufs.at[pl.ds(b * CK, CK)]
                pltpu.make_async_copy(
                    ta.at[pl.ds(sid * 768 + j * CK, CK)], dst, gs[b]).wait()
                sd.append(pltpu.async_copy(dst, accb.at[bat_v.at[j]],
                                           ss[b], add=True))

                @pl.when(cid == 0)
                def _(j=j, b=b):
                    pltpu.async_copy(ones_v, accbc.at[bat_v.at[j]],
                                     cs[b], add=True)

            for b in range(ub):
                sd[b].wait()
                j = g * ub + b

                @pl.when(cid == 0)
                def _(j=j, b=b):
                    pltpu.make_async_copy(ones_v, accbc.at[bat_v.at[j]],
                                          cs[b]).wait()

            return carry

        lax.fori_loop(0, ngb, grp_b, 0)
        plsc.subcore_barrier()
        pltpu.sync_copy(accs.at[pl.ds(sid * rzn, rzn)],
                        outs.at[pl.ds(cid * N3PAD + sid * rzn, rzn)])
        pltpu.sync_copy(accb.at[pl.ds(sid * rzb, rzb)],
                        outb.at[pl.ds(cid * BPAD + sid * rzb, rzb)])

        @pl.when(cid == 0)
        def _():
            pltpu.sync_copy(accc.at[pl.ds(sid * rzn, rzn)],
                            outc.at[pl.ds(sid * rzn, rzn)])
            pltpu.sync_copy(accbc.at[pl.ds(sid * rzb, rzb)],
                            outbc.at[pl.ds(sid * rzb, rzb)])

    return k


def _make_pool_batch(nchunks, ub=3):
    ept = nchunks * CK
    rzb = BPAD // NS
    ng = nchunks // ub

    @functools.partial(
        pl.kernel,
        out_type=(_SDS((2 * BPAD, 32), F32), _SDS((BPAD, 16), F32)),
        mesh=_MESH,
        compiler_params=_SC_PARAMS,
        scratch_types=[
            pltpu.VMEM((nchunks, CK), jnp.int32),
            pltpu.VMEM((ub * CK, 32), F32),
            pltpu.VMEM((CK, 16), F32),
            pltpu.VMEM_SHARED((BPAD, 32), F32),
            pltpu.VMEM_SHARED((BPAD, 16), F32),
        ] + [pltpu.SemaphoreType.DMA] * (3 * ub),
    )
    def k(ta, tb, coli, z32b, z16b, ones, outs, outc,
          col_v, bufs, ones_v, acc, accc, *sems):
        gs, ss, cs = sems[:ub], sems[ub:2 * ub], sems[2 * ub:]
        cid = lax.axis_index("c")
        sid = lax.axis_index("s")
        pltpu.sync_copy(z32b.at[pl.ds(sid * rzb, rzb)],
                        acc.at[pl.ds(sid * rzb, rzb)])
        pltpu.sync_copy(z16b.at[pl.ds(sid * rzb, rzb)],
                        accc.at[pl.ds(sid * rzb, rzb)])
        pltpu.sync_copy(coli.at[sid], col_v)
        pltpu.sync_copy(ones, ones_v)
        plsc.subcore_barrier()

        def grp(g, carry):
            for b in range(ub):
                j = g * ub + b
                _fire_half_gather(cid, ta, tb,
                                  pl.ds(sid * ept + j * CK, CK),
                                  bufs.at[pl.ds(b * CK, CK)], gs[b])
            sd = []
            for b in range(ub):
                j = g * ub + b
                dst = bufs.at[pl.ds(b * CK, CK)]
                pltpu.make_async_copy(
                    ta.at[pl.ds(sid * ept + j * CK, CK)], dst, gs[b]).wait()
                sd.append(pltpu.async_copy(dst, acc.at[col_v.at[j]],
                                           ss[b], add=True))

                @pl.when(cid == 0)
                def _(j=j, b=b):
                    pltpu.async_copy(ones_v, accc.at[col_v.at[j]],
                                     cs[b], add=True)

            for b in range(ub):
                sd[b].wait()
                j = g * ub + b

                @pl.when(cid == 0)
                def _(j=j, b=b):
                    pltpu.make_async_copy(ones_v, accc.at[col_v.at[j]],
                                          cs[b]).wait()

            return carry

        lax.fori_loop(0, ng, grp, 0)
        plsc.subcore_barrier()
        pltpu.sync_copy(acc.at[pl.ds(sid * rzb, rzb)],
                        outs.at[pl.ds(cid * BPAD + sid * rzb, rzb)])

        @pl.when(cid == 0)
        def _():
            pltpu.sync_copy(accc.at[pl.ds(sid * rzb, rzb)],
                            outc.at[pl.ds(sid * rzb, rzb)])

    return k


def _elu(a):
    return jnp.where(a > 0, a, jnp.exp(jnp.minimum(a, 0.0)) - 1.0)


def _make_msg(m_in_pad, m_out, be=256):
    kin = m_in_pad * 128

    def body(xg, ea, wa, ba, wb2, bb2, out):
        h = jnp.maximum(ea[...] @ wa[...] + ba[...], 0.0)
        xgv = xg[...]
        v = (xgv[:, :, None] * h[:, None, :]).reshape(be, kin)
        out[...] = (
            lax.dot_general(v, wb2[...], (((1,), (0,)), ((), ())),
                            preferred_element_type=F32)
            + xgv @ bb2[...])

    return pl.pallas_call(
        body,
        grid=(EPAD // be,),
        in_specs=[
            pl.BlockSpec((be, m_in_pad), lambda i: (i, 0)),
            pl.BlockSpec((be, 8), lambda i: (i, 0)),
            pl.BlockSpec((8, 128), lambda i: (0, 0)),
            pl.BlockSpec((1, 128), lambda i: (0, 0)),
            pl.BlockSpec((kin, m_out), lambda i: (0, 0)),
            pl.BlockSpec((m_in_pad, m_out), lambda i: (0, 0)),
        ],
        out_specs=pl.BlockSpec((be, m_out), lambda i: (i, 0)),
        out_shape=_SDS((EPAD, m_out), F32),
    )


def _make_node(m_in_pad, d, npad, split=False, bn=512):
    nb = npad // bn

    def body(p0, p1, xb, root, bias, *outs):
        a = _elu(p0[...] + p1[...] + xb[...] @ root[...] + bias[...])
        if split:
            outs[0][...] = a[:, :32]
            outs[1][...] = a[:, 32:]
        else:
            outs[0][...] = a

    if split:
        out_specs = (pl.BlockSpec((bn, 32), lambda i: (i, 0)),
                     pl.BlockSpec((bn, 32), lambda i: (i, 0)))
        out_shape = (_SDS((npad, 32), F32), _SDS((npad, 32), F32))
    else:
        out_specs = pl.BlockSpec((bn, d), lambda i: (i, 0))
        out_shape = _SDS((npad, d), F32)

    return pl.pallas_call(
        body,
        grid=(nb,),
        in_specs=[
            pl.BlockSpec((bn, d), lambda i: (i, 0)),
            pl.BlockSpec((bn, d), lambda i: (i + nb, 0)),
            pl.BlockSpec((bn, m_in_pad), lambda i: (i, 0)),
            pl.BlockSpec((m_in_pad, d), lambda i: (0, 0)),
            pl.BlockSpec((1, d), lambda i: (0, 0)),
        ],
        out_specs=out_specs,
        out_shape=out_shape,
    )


def _make_gc_pre(bn=512):
    nb = N3PAD // bn

    def body(s_lo, s_hi, c, iso, wrel_a, wrel_b, wroot_a, wroot_b, brel,
             ta_out, tb_out, r_out):
        cnt = jnp.maximum(c[...], 1.0)[:, 0:1]
        h3m = jnp.concatenate([s_lo[...], s_hi[...]], axis=1) / cnt
        isov = iso[...]
        t = h3m @ wrel_a[...] + isov @ wrel_b[...]
        ta_out[...] = t[:, :32]
        tb_out[...] = t[:, 32:]
        r_out[...] = h3m @ wroot_a[...] + isov @ wroot_b[...] + brel[...]

    return pl.pallas_call(
        body,
        grid=(nb,),
        in_specs=[
            pl.BlockSpec((bn, 32), lambda i: (i, 0)),
            pl.BlockSpec((bn, 32), lambda i: (i + nb, 0)),
            pl.BlockSpec((bn, 16), lambda i: (i, 0)),
            pl.BlockSpec((bn, 16), lambda i: (i, 0)),
            pl.BlockSpec((64, 64), lambda i: (0, 0)),
            pl.BlockSpec((16, 64), lambda i: (0, 0)),
            pl.BlockSpec((64, 64), lambda i: (0, 0)),
            pl.BlockSpec((16, 64), lambda i: (0, 0)),
            pl.BlockSpec((1, 64), lambda i: (0, 0)),
        ],
        out_specs=(pl.BlockSpec((bn, 32), lambda i: (i, 0)),
                   pl.BlockSpec((bn, 32), lambda i: (i, 0)),
                   pl.BlockSpec((bn, 64), lambda i: (i, 0))),
        out_shape=(_SDS((N3PAD, 32), F32), _SDS((N3PAD, 32), F32),
                   _SDS((N3PAD, 64), F32)),
    )


def _make_gc_mid(bn=512):
    nb = N3PAD // bn

    def body(a_lo, a_hi, r6, wrel, wroot, brel, ta_out, tb_out, r_out):
        h3b = _elu(jnp.concatenate([a_lo[...], a_hi[...]], axis=1) + r6[...])
        t = h3b @ wrel[...]
        ta_out[...] = t[:, :32]
        tb_out[...] = t[:, 32:]
        r_out[...] = h3b @ wroot[...] + brel[...]

    return pl.pallas_call(
        body,
        grid=(nb,),
        in_specs=[
            pl.BlockSpec((bn, 32), lambda i: (i, 0)),
            pl.BlockSpec((bn, 32), lambda i: (i + nb, 0)),
            pl.BlockSpec((bn, 64), lambda i: (i, 0)),
            pl.BlockSpec((64, 64), lambda i: (0, 0)),
            pl.BlockSpec((64, 64), lambda i: (0, 0)),
            pl.BlockSpec((1, 64), lambda i: (0, 0)),
        ],
        out_specs=(pl.BlockSpec((bn, 32), lambda i: (i, 0)),
                   pl.BlockSpec((bn, 32), lambda i: (i, 0)),
                   pl.BlockSpec((bn, 64), lambda i: (i, 0))),
        out_shape=(_SDS((N3PAD, 32), F32), _SDS((N3PAD, 32), F32),
                   _SDS((N3PAD, 64), F32)),
    )


def _make_gc_post(bn=512):
    nb = N3PAD // bn

    def body(a_lo, a_hi, r7, fa_out, fb_out):
        a = _elu(jnp.concatenate([a_lo[...], a_hi[...]], axis=1) + r7[...])
        fa_out[...] = a[:, :32]
        fb_out[...] = a[:, 32:]

    return pl.pallas_call(
        body,
        grid=(nb,),
        in_specs=[
            pl.BlockSpec((bn, 32), lambda i: (i, 0)),
            pl.BlockSpec((bn, 32), lambda i: (i + nb, 0)),
            pl.BlockSpec((bn, 64), lambda i: (i, 0)),
        ],
        out_specs=(pl.BlockSpec((bn, 32), lambda i: (i, 0)),
                   pl.BlockSpec((bn, 32), lambda i: (i, 0))),
        out_shape=(_SDS((N3PAD, 32), F32), _SDS((N3PAD, 32), F32)),
    )


def _make_head():
    def body(s10, s11, c1, s30, s31, c3,
             w1a, w1b, b1, w2, b2, w3r, b3, out):
        cnt1 = jnp.maximum(c1[...], 1.0)[:, 0:1]
        x1 = jnp.concatenate([s10[...], s11[...]], axis=1) / cnt1
        cnt3 = jnp.maximum(c3[...], 1.0)[:, 0:1]
        x3 = jnp.concatenate([s30[...], s31[...]], axis=1) / cnt3
        y = _elu(x1 @ w1a[...] + x3 @ w1b[...] + b1[...])
        y = _elu(y @ w2[...] + b2[...])
        out[...] = jnp.sum(y * w3r[...], axis=1, keepdims=True) + b3[...]

    bs = lambda shape: pl.BlockSpec(shape, lambda i: (0, 0))
    return pl.pallas_call(
        body,
        grid=(1,),
        in_specs=[
            pl.BlockSpec((BPAD, 32), lambda i: (0, 0)),
            pl.BlockSpec((BPAD, 32), lambda i: (1, 0)),
            bs((BPAD, 16)),
            pl.BlockSpec((BPAD, 32), lambda i: (0, 0)),
            pl.BlockSpec((BPAD, 32), lambda i: (1, 0)),
            bs((BPAD, 16)),
            bs((64, 64)), bs((64, 64)), bs((1, 64)),
            bs((64, 32)), bs((1, 32)), bs((1, 32)), bs((1, 1)),
        ],
        out_specs=pl.BlockSpec((BPAD, 1), lambda i: (0, 0)),
        out_shape=_SDS((BPAD, 1), F32),
    )


def _pad_idx(idx, n_pad, fill):
    v = jnp.full((n_pad,), fill, jnp.int32)
    return v.at[: idx.shape[0]].set(idx.astype(jnp.int32))


def _prep_nnconv(Wa, ba, Wb, bb, root, bias, m_in, m_in_pad, m_out):
    wa8 = jnp.zeros((8, 128), F32).at[:6].set(Wa)
    ba2 = ba.reshape(1, 128)
    wb3 = Wb.reshape(128, m_in, m_out).transpose(1, 0, 2)
    wb2 = jnp.zeros((m_in_pad, 128, m_out), F32).at[:m_in].set(wb3)
    wb2 = wb2.reshape(m_in_pad * 128, m_out)
    bb2 = jnp.zeros((m_in_pad, m_out), F32).at[:m_in].set(bb.reshape(m_in, m_out))
    rootp = jnp.zeros((m_in_pad, m_out), F32).at[:m_in].set(root)
    bias2 = bias.reshape(1, m_out)
    return wa8, ba2, wb2, bb2, rootp, bias2


def kernel(x, edge_index, edge_attr, batch, assignment_index_3, iso_type_3,
           edge_index_3, batch_3, W1a, b1a, W1b, b1b, root1, bias1,
           W2a, b2a, W2b, b2b, root2, bias2, W3a, b3a, W3b, b3b, root3, bias3,
           Wrel6, brel6, Wroot6, Wrel7, brel7, Wroot7,
           fc1_W, fc1_b, fc2_W, fc2_b, fc3_W, fc3_b):
    xpad = jnp.zeros((NPAD, 16), F32).at[:_N, :_F_IN].set(x)
    eapad = jnp.zeros((EPAD, 8), F32).at[:_E, :6].set(edge_attr)
    src_i = _pad_idx(edge_index[0], EPAD, 0).reshape(NW, 6, CK)
    dst_i = _pad_idx(edge_index[1], EPAD, DUM_N).reshape(NW, 6, CK)
    row3_i = _pad_idx(assignment_index_3[0], APAD, 0).reshape(NS, 44, CK)
    col3_i = _pad_idx(assignment_index_3[1], APAD, DUM_N3).reshape(NS, 44, CK)
    src3_i = _pad_idx(edge_index_3[0], E3PAD, 0).reshape(NS, 60, CK)
    dst3_i = _pad_idx(edge_index_3[1], E3PAD, DUM_N3).reshape(NS, 60, CK)
    batch_i = _pad_idx(batch, NPAD, DUM_B).reshape(NS, 6, CK)
    batch3_i = _pad_idx(batch_3, N3PAD, DUM_B).reshape(NS, 15, CK)
    isopad = jnp.zeros((N3PAD, 16), F32).at[:_N3].set(iso_type_3)

    zN32 = jnp.zeros((NPAD, 32), F32)
    zN64 = jnp.zeros((NPAD, 64), F32)
    z32N3 = jnp.zeros((N3PAD, 32), F32)
    z16N3 = jnp.zeros((N3PAD, 16), F32)
    z32B = jnp.zeros((BPAD, 32), F32)
    z16B = jnp.zeros((BPAD, 16), F32)
    ones128 = jnp.ones((CK, 16), F32)

    p1w = _prep_nnconv(W1a, b1a, W1b, b1b, root1, bias1, _F_IN, 16, 32)
    p2w = _prep_nnconv(W2a, b2a, W2b, b2b, root2, bias2, 32, 32, 64)
    p3w = _prep_nnconv(W3a, b3a, W3b, b3b, root3, bias3, 64, 64, 64)

    xg1 = _make_gather(16, 6)(xpad, src_i)
    msg1 = _make_msg(16, 32)(xg1, eapad, *p1w[:4])
    agg1 = _make_scatter(32, 6, NPAD)(msg1, dst_i, zN32)
    h1 = _make_node(16, 32, NPAD)(agg1, agg1, xpad, p1w[4], p1w[5])

    xg2 = _make_gather(32, 6)(h1, src_i)
    msg2 = _make_msg(32, 64)(xg2, eapad, *p2w[:4])
    agg2 = _make_scatter(64, 6, NPAD)(msg2, dst_i, zN64)
    h2 = _make_node(32, 64, NPAD)(agg2, agg2, h1, p2w[4], p2w[5])

    xg3 = _make_gather(64, 6)(h2, src_i)
    msg3 = _make_msg(64, 64)(xg3, eapad, *p3w[:4])
    agg3 = _make_scatter(64, 6, NPAD)(msg3, dst_i, zN64)
    ha, hb = _make_node(64, 64, NPAD, split=True)(agg3, agg3, h2, p3w[4], p3w[5])

    s3sum, ccol, s1p, c1p = _make_pool3()(
        ha, hb, row3_i, col3_i, batch_i, z32N3, z16N3, z32B, z16B, ones128)

    t6a, t6b, r6 = _make_gc_pre()(
        s3sum, s3sum, ccol, isopad,
        Wrel6[:64], Wrel6[64:], Wroot6[:64], Wroot6[64:], brel6.reshape(1, 64))
    agg6 = _make_pool_split(60, N3PAD)(t6a, t6b, src3_i, dst3_i, z32N3)
    t7a, t7b, r7 = _make_gc_mid()(
        agg6, agg6, r6, Wrel7, Wroot7, brel7.reshape(1, 64))
    agg7 = _make_pool_split(60, N3PAD)(t7a, t7b, src3_i, dst3_i, z32N3)
    fa, fb = _make_gc_post()(agg7, agg7, r7)
    s3p, c3p = _make_pool_batch(15)(fa, fb, batch3_i, z32B, z16B, ones128)

    out = _make_head()(
        s1p, s1p, c1p, s3p, s3p, c3p,
        fc1_W[:64], fc1_W[64:], fc1_b.reshape(1, 64),
        fc2_W, fc2_b.reshape(1, 32),
        fc3_W.reshape(1, 32), fc3_b.reshape(1, 1))
    return out[:_B, 0]


# speedup vs baseline: 2.5272x; 1.0365x over previous
import functools

import jax
import jax.numpy as jnp
from jax import lax
from jax.experimental import pallas as pl
from jax.experimental.pallas import tpu as pltpu
from jax.experimental.pallas import tpu_sc as plsc

F32 = jnp.float32

_N = 12000
_E = 24000
_B = 1024
_N3 = 30000
_A = 90000
_E3 = 120000
_F_IN = 13
_NI3 = 16

NC, NS = 2, 16
NW = NC * NS
CK = 128

NPAD = 12288
EPAD = 24576
N3PAD = 30720
APAD = 90112
E3PAD = 122880
BPAD = 1280

DUM_N = NPAD - 8
DUM_N3 = N3PAD - 8
DUM_B = BPAD - 8

_MESH = plsc.VectorSubcoreMesh(
    core_axis_name="c", subcore_axis_name="s", num_cores=NC, num_subcores=NS)
_SC_PARAMS = pltpu.CompilerParams(use_tc_tiling_on_sc=False)
_SDS = jax.ShapeDtypeStruct


def _make_gather(d, nchunks):
    ept = nchunks * CK

    @functools.partial(
        pl.kernel,
        out_type=_SDS((NW * ept, d), F32),
        mesh=_MESH,
        compiler_params=_SC_PARAMS,
        scratch_types=[
            pltpu.VMEM((nchunks, CK), jnp.int32),
            pltpu.VMEM((ept, d), F32),
            pltpu.SemaphoreType.DMA,
        ],
    )
    def k(table, idx, out, idx_v, buf, sem):
        cid = lax.axis_index("c")
        sid = lax.axis_index("s")
        wid = sid * NC + cid
        pltpu.sync_copy(idx.at[wid], idx_v)
        descs = [
            pltpu.async_copy(table.at[idx_v.at[j]],
                             buf.at[pl.ds(j * CK, CK)], sem)
            for j in range(nchunks)
        ]
        for dsc in descs:
            dsc.wait()
        pltpu.sync_copy(buf, out.at[pl.ds(wid * ept, ept)])

    return k


def _make_scatter(d, nchunks, rows, u=3):
    ept = nchunks * CK
    rz = rows // NS
    ng = nchunks // u

    @functools.partial(
        pl.kernel,
        out_type=_SDS((2 * rows, d), F32),
        mesh=_MESH,
        compiler_params=_SC_PARAMS,
        scratch_types=[
            pltpu.VMEM((nchunks, CK), jnp.int32),
            pltpu.VMEM((u * CK, d), F32),
            pltpu.VMEM_SHARED((rows, d), F32),
        ] + [pltpu.SemaphoreType.DMA] * (2 * u),
    )
    def k(data, idx, zeros, out, idx_v, bufs, acc, *sems):
        gs, ss = sems[:u], sems[u:]
        cid = lax.axis_index("c")
        sid = lax.axis_index("s")
        wid = sid * NC + cid
        pltpu.sync_copy(zeros.at[pl.ds(sid * rz, rz)], acc.at[pl.ds(sid * rz, rz)])
        pltpu.sync_copy(idx.at[wid], idx_v)
        plsc.subcore_barrier()

        def grp(g, carry):
            gd = []
            for b in range(u):
                j = g * u + b
                gd.append(pltpu.async_copy(
                    data.at[pl.ds(wid * ept + j * CK, CK)],
                    bufs.at[pl.ds(b * CK, CK)], gs[b]))
            sd = []
            for b in range(u):
                j = g * u + b
                gd[b].wait()
                sd.append(pltpu.async_copy(
                    bufs.at[pl.ds(b * CK, CK)], acc.at[idx_v.at[j]],
                    ss[b], add=True))
            for dsc in sd:
                dsc.wait()
            return carry

        lax.fori_loop(0, ng, grp, 0)
        plsc.subcore_barrier()
        pltpu.sync_copy(acc.at[pl.ds(sid * rz, rz)],
                        out.at[pl.ds(cid * rows + sid * rz, rz)])

    return k


def _fire_half_gather(cid, ta, tb, idx_row, dst, sem):
    @pl.when(cid == 0)
    def _():
        pltpu.async_copy(ta.at[idx_row], dst, sem)

    @pl.when(cid == 1)
    def _():
        pltpu.async_copy(tb.at[idx_row], dst, sem)


def _make_pool_split(nchunks, rows, u=4):
    rz = rows // NS
    ng = nchunks // u

    @functools.partial(
        pl.kernel,
        out_type=_SDS((2 * rows, 32), F32),
        mesh=_MESH,
        compiler_params=_SC_PARAMS,
        scratch_types=[
            pltpu.VMEM((nchunks, CK), jnp.int32),
            pltpu.VMEM((nchunks, CK), jnp.int32),
            pltpu.VMEM((u * CK, 32), F32),
            pltpu.VMEM_SHARED((rows, 32), F32),
        ] + [pltpu.SemaphoreType.DMA] * (2 * u),
    )
    def k(ta, tb, rowi, coli, zeros, out, row_v, col_v, bufs, acc, *sems):
        gs, ss = sems[:u], sems[u:]
        cid = lax.axis_index("c")
        sid = lax.axis_index("s")
        pltpu.sync_copy(zeros.at[pl.ds(sid * rz, rz)], acc.at[pl.ds(sid * rz, rz)])
        pltpu.sync_copy(rowi.at[sid], row_v)
        pltpu.sync_copy(coli.at[sid], col_v)
        plsc.subcore_barrier()

        def grp(g, carry):
            for b in range(u):
                j = g * u + b
                _fire_half_gather(cid, ta, tb, row_v.at[j],
                                  bufs.at[pl.ds(b * CK, CK)], gs[b])
            sd = []
            for b in range(u):
                j = g * u + b
                dst = bufs.at[pl.ds(b * CK, CK)]
                pltpu.make_async_copy(ta.at[row_v.at[j]], dst, gs[b]).wait()
                sd.append(pltpu.async_copy(dst, acc.at[col_v.at[j]],
                                           ss[b], add=True))
            for dsc in sd:
                dsc.wait()
            return carry

        lax.fori_loop(0, ng, grp, 0)
        plsc.subcore_barrier()
        pltpu.sync_copy(acc.at[pl.ds(sid * rz, rz)],
                        out.at[pl.ds(cid * rows + sid * rz, rz)])

    return k


def _make_pool3(u=4, ub=3):
    rzn = N3PAD // NS
    rzb = BPAD // NS
    nga = 44 // u
    ngb = 6 // ub

    @functools.partial(
        pl.kernel,
        out_type=(_SDS((2 * N3PAD, 32), F32), _SDS((N3PAD, 16), F32),
                  _SDS((2 * BPAD, 32), F32), _SDS((BPAD, 16), F32)),
        mesh=_MESH,
        compiler_params=_SC_PARAMS,
        scratch_types=[
            pltpu.VMEM((44, CK), jnp.int32),
            pltpu.VMEM((44, CK), jnp.int32),
            pltpu.VMEM((6, CK), jnp.int32),
            pltpu.VMEM((u * CK, 32), F32),
            pltpu.VMEM((CK, 16), F32),
            pltpu.VMEM_SHARED((N3PAD, 32), F32),
            pltpu.VMEM_SHARED((N3PAD, 16), F32),
            pltpu.VMEM_SHARED((BPAD, 32), F32),
            pltpu.VMEM_SHARED((BPAD, 16), F32),
        ] + [pltpu.SemaphoreType.DMA] * (3 * u),
    )
    def k(ta, tb, rowi, coli, bati, z32n, z16n, z32b, z16b, ones,
          outs, outc, outb, outbc,
          row_v, col_v, bat_v, bufs, ones_v, accs, accc, accb, accbc, *sems):
        gs, ss, cs = sems[:u], sems[u:2 * u], sems[2 * u:]
        cid = lax.axis_index("c")
        sid = lax.axis_index("s")
        pltpu.sync_copy(z32n.at[pl.ds(sid * rzn, rzn)],
                        accs.at[pl.ds(sid * rzn, rzn)])
        pltpu.sync_copy(z16n.at[pl.ds(sid * rzn, rzn)],
                        accc.at[pl.ds(sid * rzn, rzn)])
        pltpu.sync_copy(z32b.at[pl.ds(sid * rzb, rzb)],
                        accb.at[pl.ds(sid * rzb, rzb)])
        pltpu.sync_copy(z16b.at[pl.ds(sid * rzb, rzb)],
                        accbc.at[pl.ds(sid * rzb, rzb)])
        pltpu.sync_copy(rowi.at[sid], row_v)
        pltpu.sync_copy(coli.at[sid], col_v)
        pltpu.sync_copy(bati.at[sid], bat_v)
        pltpu.sync_copy(ones, ones_v)
        plsc.subcore_barrier()

        def grp_a(g, carry):
            for b in range(u):
                j = g * u + b
                _fire_half_gather(cid, ta, tb, row_v.at[j],
                                  bufs.at[pl.ds(b * CK, CK)], gs[b])
            sd = []
            for b in range(u):
                j = g * u + b
                dst = bufs.at[pl.ds(b * CK, CK)]
                pltpu.make_async_copy(ta.at[row_v.at[j]], dst, gs[b]).wait()
                sd.append(pltpu.async_copy(dst, accs.at[col_v.at[j]],
                                           ss[b], add=True))

                @pl.when(cid == 0)
                def _(j=j, b=b):
                    pltpu.async_copy(ones_v, accc.at[col_v.at[j]],
                                     cs[b], add=True)

            for b in range(u):
                sd[b].wait()
                j = g * u + b

                @pl.when(cid == 0)
                def _(j=j, b=b):
                    pltpu.make_async_copy(ones_v, accc.at[col_v.at[j]],
                                          cs[b]).wait()

            return carry

        lax.fori_loop(0, nga, grp_a, 0)

        def grp_b(g, carry):
            for b in range(ub):
                j = g * ub + b
                _fire_half_gather(cid, ta, tb,
                                  pl.ds(sid * 768 + j * CK, CK),
                                  bufs.at[pl.ds(b * CK, CK)], gs[b])
            sd = []
            for b in range(ub):
                j = g * ub + b
                dst = bufs.at[pl.ds(b * CK, CK)]
                pltpu.make_async_copy(
                    ta.at[pl.ds(sid * 768 + j * CK, CK)], dst, gs[b]).wait()
                sd.append(pltpu.async_copy(dst, accb.at[bat_v.at[j]],
                                           ss[b], add=True))

                @pl.when(cid == 0)
                def _(j=j, b=b):
                    pltpu.async_copy(ones_v, accbc.at[bat_v.at[j]],
                                     cs[b], add=True)

            for b in range(ub):
                sd[b].wait()
                j = g * ub + b

                @pl.when(cid == 0)
                def _(j=j, b=b):
                    pltpu.make_async_copy(ones_v, accbc.at[bat_v.at[j]],
                                          cs[b]).wait()

            return carry

        lax.fori_loop(0, ngb, grp_b, 0)
        plsc.subcore_barrier()
        pltpu.sync_copy(accs.at[pl.ds(sid * rzn, rzn)],
                        outs.at[pl.ds(cid * N3PAD + sid * rzn, rzn)])
        pltpu.sync_copy(accb.at[pl.ds(sid * rzb, rzb)],
                        outb.at[pl.ds(cid * BPAD + sid * rzb, rzb)])

        @pl.when(cid == 0)
        def _():
            pltpu.sync_copy(accc.at[pl.ds(sid * rzn, rzn)],
                            outc.at[pl.ds(sid * rzn, rzn)])
            pltpu.sync_copy(accbc.at[pl.ds(sid * rzb, rzb)],
                            outbc.at[pl.ds(sid * rzb, rzb)])

    return k


def _make_pool_batch(nchunks, ub=3):
    ept = nchunks * CK
    rzb = BPAD // NS
    ng = nchunks // ub

    @functools.partial(
        pl.kernel,
        out_type=(_SDS((2 * BPAD, 32), F32), _SDS((BPAD, 16), F32)),
        mesh=_MESH,
        compiler_params=_SC_PARAMS,
        scratch_types=[
            pltpu.VMEM((nchunks, CK), jnp.int32),
            pltpu.VMEM((ub * CK, 32), F32),
            pltpu.VMEM((CK, 16), F32),
            pltpu.VMEM_SHARED((BPAD, 32), F32),
            pltpu.VMEM_SHARED((BPAD, 16), F32),
        ] + [pltpu.SemaphoreType.DMA] * (3 * ub),
    )
    def k(ta, tb, coli, z32b, z16b, ones, outs, outc,
          col_v, bufs, ones_v, acc, accc, *sems):
        gs, ss, cs = sems[:ub], sems[ub:2 * ub], sems[2 * ub:]
        cid = lax.axis_index("c")
        sid = lax.axis_index("s")
        pltpu.sync_copy(z32b.at[pl.ds(sid * rzb, rzb)],
                        acc.at[pl.ds(sid * rzb, rzb)])
        pltpu.sync_copy(z16b.at[pl.ds(sid * rzb, rzb)],
                        accc.at[pl.ds(sid * rzb, rzb)])
        pltpu.sync_copy(coli.at[sid], col_v)
        pltpu.sync_copy(ones, ones_v)
        plsc.subcore_barrier()

        def grp(g, carry):
            for b in range(ub):
                j = g * ub + b
                _fire_half_gather(cid, ta, tb,
                                  pl.ds(sid * ept + j * CK, CK),
                                  bufs.at[pl.ds(b * CK, CK)], gs[b])
            sd = []
            for b in range(ub):
                j = g * ub + b
                dst = bufs.at[pl.ds(b * CK, CK)]
                pltpu.make_async_copy(
                    ta.at[pl.ds(sid * ept + j * CK, CK)], dst, gs[b]).wait()
                sd.append(pltpu.async_copy(dst, acc.at[col_v.at[j]],
                                           ss[b], add=True))

                @pl.when(cid == 0)
                def _(j=j, b=b):
                    pltpu.async_copy(ones_v, accc.at[col_v.at[j]],
                                     cs[b], add=True)

            for b in range(ub):
                sd[b].wait()
                j = g * ub + b

                @pl.when(cid == 0)
                def _(j=j, b=b):
                    pltpu.make_async_copy(ones_v, accc.at[col_v.at[j]],
                                          cs[b]).wait()

            return carry

        lax.fori_loop(0, ng, grp, 0)
        plsc.subcore_barrier()
        pltpu.sync_copy(acc.at[pl.ds(sid * rzb, rzb)],
                        outs.at[pl.ds(cid * BPAD + sid * rzb, rzb)])

        @pl.when(cid == 0)
        def _():
            pltpu.sync_copy(accc.at[pl.ds(sid * rzb, rzb)],
                            outc.at[pl.ds(sid * rzb, rzb)])

    return k


def _elu(a):
    return jnp.where(a > 0, a, jnp.exp(jnp.minimum(a, 0.0)) - 1.0)


def _make_msg(m_in_pad, m_out, be=256):
    kin = m_in_pad * 128

    def body(xg, ea, wa, ba, wb2, bb2, out):
        h = jnp.maximum(ea[...] @ wa[...] + ba[...], 0.0)
        xgv = xg[...]
        v = jnp.concatenate(
            [(xgv[:, i:i + 1] * h).astype(jnp.bfloat16)
             for i in range(m_in_pad)], axis=1)
        out[...] = (
            lax.dot_general(v, wb2[...], (((1,), (0,)), ((), ())),
                            preferred_element_type=F32)
            + xgv @ bb2[...])

    return pl.pallas_call(
        body,
        grid=(EPAD // be,),
        in_specs=[
            pl.BlockSpec((be, m_in_pad), lambda i: (i, 0)),
            pl.BlockSpec((be, 8), lambda i: (i, 0)),
            pl.BlockSpec((8, 128), lambda i: (0, 0)),
            pl.BlockSpec((1, 128), lambda i: (0, 0)),
            pl.BlockSpec((kin, m_out), lambda i: (0, 0)),
            pl.BlockSpec((m_in_pad, m_out), lambda i: (0, 0)),
        ],
        out_specs=pl.BlockSpec((be, m_out), lambda i: (i, 0)),
        out_shape=_SDS((EPAD, m_out), F32),
    )


def _make_node(m_in_pad, d, npad, split=False, bn=512):
    nb = npad // bn

    def body(p0, p1, xb, root, bias, *outs):
        a = _elu(p0[...] + p1[...] + xb[...] @ root[...] + bias[...])
        if split:
            outs[0][...] = a[:, :32]
            outs[1][...] = a[:, 32:]
        else:
            outs[0][...] = a

    if split:
        out_specs = (pl.BlockSpec((bn, 32), lambda i: (i, 0)),
                     pl.BlockSpec((bn, 32), lambda i: (i, 0)))
        out_shape = (_SDS((npad, 32), F32), _SDS((npad, 32), F32))
    else:
        out_specs = pl.BlockSpec((bn, d), lambda i: (i, 0))
        out_shape = _SDS((npad, d), F32)

    return pl.pallas_call(
        body,
        grid=(nb,),
        in_specs=[
            pl.BlockSpec((bn, d), lambda i: (i, 0)),
            pl.BlockSpec((bn, d), lambda i: (i + nb, 0)),
            pl.BlockSpec((bn, m_in_pad), lambda i: (i, 0)),
            pl.BlockSpec((m_in_pad, d), lambda i: (0, 0)),
            pl.BlockSpec((1, d), lambda i: (0, 0)),
        ],
        out_specs=out_specs,
        out_shape=out_shape,
    )


def _make_gc_pre(bn=512):
    nb = N3PAD // bn

    def body(s_lo, s_hi, c, iso, wrel_a, wrel_b, wroot_a, wroot_b, brel,
             ta_out, tb_out, r_out):
        cnt = jnp.maximum(c[...], 1.0)[:, 0:1]
        h3m = jnp.concatenate([s_lo[...], s_hi[...]], axis=1) / cnt
        isov = iso[...]
        t = h3m @ wrel_a[...] + isov @ wrel_b[...]
        ta_out[...] = t[:, :32]
        tb_out[...] = t[:, 32:]
        r_out[...] = h3m @ wroot_a[...] + isov @ wroot_b[...] + brel[...]

    return pl.pallas_call(
        body,
        grid=(nb,),
        in_specs=[
            pl.BlockSpec((bn, 32), lambda i: (i, 0)),
            pl.BlockSpec((bn, 32), lambda i: (i + nb, 0)),
            pl.BlockSpec((bn, 16), lambda i: (i, 0)),
            pl.BlockSpec((bn, 16), lambda i: (i, 0)),
            pl.BlockSpec((64, 64), lambda i: (0, 0)),
            pl.BlockSpec((16, 64), lambda i: (0, 0)),
            pl.BlockSpec((64, 64), lambda i: (0, 0)),
            pl.BlockSpec((16, 64), lambda i: (0, 0)),
            pl.BlockSpec((1, 64), lambda i: (0, 0)),
        ],
        out_specs=(pl.BlockSpec((bn, 32), lambda i: (i, 0)),
                   pl.BlockSpec((bn, 32), lambda i: (i, 0)),
                   pl.BlockSpec((bn, 64), lambda i: (i, 0))),
        out_shape=(_SDS((N3PAD, 32), F32), _SDS((N3PAD, 32), F32),
                   _SDS((N3PAD, 64), F32)),
    )


def _make_gc_mid(bn=512):
    nb = N3PAD // bn

    def body(a_lo, a_hi, r6, wrel, wroot, brel, ta_out, tb_out, r_out):
        h3b = _elu(jnp.concatenate([a_lo[...], a_hi[...]], axis=1) + r6[...])
        t = h3b @ wrel[...]
        ta_out[...] = t[:, :32]
        tb_out[...] = t[:, 32:]
        r_out[...] = h3b @ wroot[...] + brel[...]

    return pl.pallas_call(
        body,
        grid=(nb,),
        in_specs=[
            pl.BlockSpec((bn, 32), lambda i: (i, 0)),
            pl.BlockSpec((bn, 32), lambda i: (i + nb, 0)),
            pl.BlockSpec((bn, 64), lambda i: (i, 0)),
            pl.BlockSpec((64, 64), lambda i: (0, 0)),
            pl.BlockSpec((64, 64), lambda i: (0, 0)),
            pl.BlockSpec((1, 64), lambda i: (0, 0)),
        ],
        out_specs=(pl.BlockSpec((bn, 32), lambda i: (i, 0)),
                   pl.BlockSpec((bn, 32), lambda i: (i, 0)),
                   pl.BlockSpec((bn, 64), lambda i: (i, 0))),
        out_shape=(_SDS((N3PAD, 32), F32), _SDS((N3PAD, 32), F32),
                   _SDS((N3PAD, 64), F32)),
    )


def _make_gc_post(bn=512):
    nb = N3PAD // bn

    def body(a_lo, a_hi, r7, fa_out, fb_out):
        a = _elu(jnp.concatenate([a_lo[...], a_hi[...]], axis=1) + r7[...])
        fa_out[...] = a[:, :32]
        fb_out[...] = a[:, 32:]

    return pl.pallas_call(
        body,
        grid=(nb,),
        in_specs=[
            pl.BlockSpec((bn, 32), lambda i: (i, 0)),
            pl.BlockSpec((bn, 32), lambda i: (i + nb, 0)),
            pl.BlockSpec((bn, 64), lambda i: (i, 0)),
        ],
        out_specs=(pl.BlockSpec((bn, 32), lambda i: (i, 0)),
                   pl.BlockSpec((bn, 32), lambda i: (i, 0))),
        out_shape=(_SDS((N3PAD, 32), F32), _SDS((N3PAD, 32), F32)),
    )


def _make_head():
    def body(s10, s11, c1, s30, s31, c3,
             w1a, w1b, b1, w2, b2, w3r, b3, out):
        cnt1 = jnp.maximum(c1[...], 1.0)[:, 0:1]
        x1 = jnp.concatenate([s10[...], s11[...]], axis=1) / cnt1
        cnt3 = jnp.maximum(c3[...], 1.0)[:, 0:1]
        x3 = jnp.concatenate([s30[...], s31[...]], axis=1) / cnt3
        y = _elu(x1 @ w1a[...] + x3 @ w1b[...] + b1[...])
        y = _elu(y @ w2[...] + b2[...])
        out[...] = jnp.sum(y * w3r[...], axis=1, keepdims=True) + b3[...]

    bs = lambda shape: pl.BlockSpec(shape, lambda i: (0, 0))
    return pl.pallas_call(
        body,
        grid=(1,),
        in_specs=[
            pl.BlockSpec((BPAD, 32), lambda i: (0, 0)),
            pl.BlockSpec((BPAD, 32), lambda i: (1, 0)),
            bs((BPAD, 16)),
            pl.BlockSpec((BPAD, 32), lambda i: (0, 0)),
            pl.BlockSpec((BPAD, 32), lambda i: (1, 0)),
            bs((BPAD, 16)),
            bs((64, 64)), bs((64, 64)), bs((1, 64)),
            bs((64, 32)), bs((1, 32)), bs((1, 32)), bs((1, 1)),
        ],
        out_specs=pl.BlockSpec((BPAD, 1), lambda i: (0, 0)),
        out_shape=_SDS((BPAD, 1), F32),
    )


def _pad_idx(idx, n_pad, fill):
    v = jnp.full((n_pad,), fill, jnp.int32)
    return v.at[: idx.shape[0]].set(idx.astype(jnp.int32))


def _prep_nnconv(Wa, ba, Wb, bb, root, bias, m_in, m_in_pad, m_out):
    wa8 = jnp.zeros((8, 128), F32).at[:6].set(Wa)
    ba2 = ba.reshape(1, 128)
    wb3 = Wb.reshape(128, m_in, m_out).transpose(1, 0, 2)
    wb2 = jnp.zeros((m_in_pad, 128, m_out), F32).at[:m_in].set(wb3)
    wb2 = wb2.reshape(m_in_pad * 128, m_out).astype(jnp.bfloat16)
    bb2 = jnp.zeros((m_in_pad, m_out), F32).at[:m_in].set(bb.reshape(m_in, m_out))
    rootp = jnp.zeros((m_in_pad, m_out), F32).at[:m_in].set(root)
    bias2 = bias.reshape(1, m_out)
    return wa8, ba2, wb2, bb2, rootp, bias2


def kernel(x, edge_index, edge_attr, batch, assignment_index_3, iso_type_3,
           edge_index_3, batch_3, W1a, b1a, W1b, b1b, root1, bias1,
           W2a, b2a, W2b, b2b, root2, bias2, W3a, b3a, W3b, b3b, root3, bias3,
           Wrel6, brel6, Wroot6, Wrel7, brel7, Wroot7,
           fc1_W, fc1_b, fc2_W, fc2_b, fc3_W, fc3_b):
    xpad = jnp.zeros((NPAD, 16), F32).at[:_N, :_F_IN].set(x)
    eapad = jnp.zeros((EPAD, 8), F32).at[:_E, :6].set(edge_attr)
    src_i = _pad_idx(edge_index[0], EPAD, 0).reshape(NW, 6, CK)
    dst_i = _pad_idx(edge_index[1], EPAD, DUM_N).reshape(NW, 6, CK)
    row3_i = _pad_idx(assignment_index_3[0], APAD, 0).reshape(NS, 44, CK)
    col3_i = _pad_idx(assignment_index_3[1], APAD, DUM_N3).reshape(NS, 44, CK)
    src3_i = _pad_idx(edge_index_3[0], E3PAD, 0).reshape(NS, 60, CK)
    dst3_i = _pad_idx(edge_index_3[1], E3PAD, DUM_N3).reshape(NS, 60, CK)
    batch_i = _pad_idx(batch, NPAD, DUM_B).reshape(NS, 6, CK)
    batch3_i = _pad_idx(batch_3, N3PAD, DUM_B).reshape(NS, 15, CK)
    isopad = jnp.zeros((N3PAD, 16), F32).at[:_N3].set(iso_type_3)

    zN32 = jnp.zeros((NPAD, 32), F32)
    zN64 = jnp.zeros((NPAD, 64), F32)
    z32N3 = jnp.zeros((N3PAD, 32), F32)
    z16N3 = jnp.zeros((N3PAD, 16), F32)
    z32B = jnp.zeros((BPAD, 32), F32)
    z16B = jnp.zeros((BPAD, 16), F32)
    ones128 = jnp.ones((CK, 16), F32)

    p1w = _prep_nnconv(W1a, b1a, W1b, b1b, root1, bias1, _F_IN, 16, 32)
    p2w = _prep_nnconv(W2a, b2a, W2b, b2b, root2, bias2, 32, 32, 64)
    p3w = _prep_nnconv(W3a, b3a, W3b, b3b, root3, bias3, 64, 64, 64)

    xg1 = _make_gather(16, 6)(xpad, src_i)
    msg1 = _make_msg(16, 32)(xg1, eapad, *p1w[:4])
    agg1 = _make_scatter(32, 6, NPAD)(msg1, dst_i, zN32)
    h1 = _make_node(16, 32, NPAD)(agg1, agg1, xpad, p1w[4], p1w[5])

    xg2 = _make_gather(32, 6)(h1, src_i)
    msg2 = _make_msg(32, 64)(xg2, eapad, *p2w[:4])
    agg2 = _make_scatter(64, 6, NPAD)(msg2, dst_i, zN64)
    h2 = _make_node(32, 64, NPAD)(agg2, agg2, h1, p2w[4], p2w[5])

    xg3 = _make_gather(64, 6)(h2, src_i)
    msg3 = _make_msg(64, 64)(xg3, eapad, *p3w[:4])
    agg3 = _make_scatter(64, 6, NPAD)(msg3, dst_i, zN64)
    ha, hb = _make_node(64, 64, NPAD, split=True)(agg3, agg3, h2, p3w[4], p3w[5])

    s3sum, ccol, s1p, c1p = _make_pool3()(
        ha, hb, row3_i, col3_i, batch_i, z32N3, z16N3, z32B, z16B, ones128)

    t6a, t6b, r6 = _make_gc_pre()(
        s3sum, s3sum, ccol, isopad,
        Wrel6[:64], Wrel6[64:], Wroot6[:64], Wroot6[64:], brel6.reshape(1, 64))
    agg6 = _make_pool_split(60, N3PAD)(t6a, t6b, src3_i, dst3_i, z32N3)
    t7a, t7b, r7 = _make_gc_mid()(
        agg6, agg6, r6, Wrel7, Wroot7, brel7.reshape(1, 64))
    agg7 = _make_pool_split(60, N3PAD)(t7a, t7b, src3_i, dst3_i, z32N3)
    fa, fb = _make_gc_post()(agg7, agg7, r7)
    s3p, c3p = _make_pool_batch(15)(fa, fb, batch3_i, z32B, z16B, ones128)

    out = _make_head()(
        s1p, s1p, c1p, s3p, s3p, c3p,
        fc1_W[:64], fc1_W[64:], fc1_b.reshape(1, 64),
        fc2_W, fc2_b.reshape(1, 32),
        fc3_W.reshape(1, 32), fc3_b.reshape(1, 1))
    return out[:_B, 0]


# speedup vs baseline: 2.8844x; 1.1413x over previous
import functools

import jax
import jax.numpy as jnp
from jax import lax
from jax.experimental import pallas as pl
from jax.experimental.pallas import tpu as pltpu
from jax.experimental.pallas import tpu_sc as plsc

F32 = jnp.float32

_N = 12000
_E = 24000
_B = 1024
_N3 = 30000
_A = 90000
_E3 = 120000
_F_IN = 13
_NI3 = 16

NC, NS = 2, 16
NW = NC * NS
CK = 128

NPAD = 12288
EPAD = 24576
N3PAD = 30720
APAD = 90112
E3PAD = 122880
BPAD = 1280

DUM_N = NPAD - 8
DUM_N3 = N3PAD - 8
DUM_B = BPAD - 8

_MESH = plsc.VectorSubcoreMesh(
    core_axis_name="c", subcore_axis_name="s", num_cores=NC, num_subcores=NS)
_SC_PARAMS = pltpu.CompilerParams(use_tc_tiling_on_sc=False)
_SDS = jax.ShapeDtypeStruct


def _make_gather(d, nchunks):
    ept = nchunks * CK

    @functools.partial(
        pl.kernel,
        out_type=_SDS((NW * ept, d), F32),
        mesh=_MESH,
        compiler_params=_SC_PARAMS,
        scratch_types=[
            pltpu.VMEM((nchunks, CK), jnp.int32),
            pltpu.VMEM((ept, d), F32),
            pltpu.SemaphoreType.DMA,
        ],
    )
    def k(table, idx, out, idx_v, buf, sem):
        cid = lax.axis_index("c")
        sid = lax.axis_index("s")
        wid = sid * NC + cid
        pltpu.sync_copy(idx.at[wid], idx_v)
        descs = [
            pltpu.async_copy(table.at[idx_v.at[j]],
                             buf.at[pl.ds(j * CK, CK)], sem)
            for j in range(nchunks)
        ]
        for dsc in descs:
            dsc.wait()
        pltpu.sync_copy(buf, out.at[pl.ds(wid * ept, ept)])

    return k


def _make_scatter(d, nchunks, rows, u=6):
    ept = nchunks * CK
    rz = rows // NS
    ng = nchunks // u

    @functools.partial(
        pl.kernel,
        out_type=_SDS((2 * rows, d), F32),
        mesh=_MESH,
        compiler_params=_SC_PARAMS,
        scratch_types=[
            pltpu.VMEM((nchunks, CK), jnp.int32),
            pltpu.VMEM((u * CK, d), F32),
            pltpu.VMEM_SHARED((rows, d), F32),
        ] + [pltpu.SemaphoreType.DMA] * (2 * u),
    )
    def k(data, idx, zeros, out, idx_v, bufs, acc, *sems):
        gs, ss = sems[:u], sems[u:]
        cid = lax.axis_index("c")
        sid = lax.axis_index("s")
        wid = sid * NC + cid
        pltpu.sync_copy(zeros.at[pl.ds(sid * rz, rz)], acc.at[pl.ds(sid * rz, rz)])
        pltpu.sync_copy(idx.at[wid], idx_v)
        plsc.subcore_barrier()

        def grp(g, carry):
            gd = []
            for b in range(u):
                j = g * u + b
                gd.append(pltpu.async_copy(
                    data.at[pl.ds(wid * ept + j * CK, CK)],
                    bufs.at[pl.ds(b * CK, CK)], gs[b]))
            sd = []
            for b in range(u):
                j = g * u + b
                gd[b].wait()
                sd.append(pltpu.async_copy(
                    bufs.at[pl.ds(b * CK, CK)], acc.at[idx_v.at[j]],
                    ss[b], add=True))
            for dsc in sd:
                dsc.wait()
            return carry

        lax.fori_loop(0, ng, grp, 0)
        plsc.subcore_barrier()
        pltpu.sync_copy(acc.at[pl.ds(sid * rz, rz)],
                        out.at[pl.ds(cid * rows + sid * rz, rz)])

    return k


def _fire_half_gather(cid, ta, tb, idx_row, dst, sem):
    @pl.when(cid == 0)
    def _():
        pltpu.async_copy(ta.at[idx_row], dst, sem)

    @pl.when(cid == 1)
    def _():
        pltpu.async_copy(tb.at[idx_row], dst, sem)


def _make_pool_split(nchunks, rows, u=6):
    rz = rows // NS
    ng = nchunks // u

    @functools.partial(
        pl.kernel,
        out_type=_SDS((2 * rows, 32), F32),
        mesh=_MESH,
        compiler_params=_SC_PARAMS,
        scratch_types=[
            pltpu.VMEM((nchunks, CK), jnp.int32),
            pltpu.VMEM((nchunks, CK), jnp.int32),
            pltpu.VMEM((u * CK, 32), F32),
            pltpu.VMEM_SHARED((rows, 32), F32),
        ] + [pltpu.SemaphoreType.DMA] * (2 * u),
    )
    def k(ta, tb, rowi, coli, zeros, out, row_v, col_v, bufs, acc, *sems):
        gs, ss = sems[:u], sems[u:]
        cid = lax.axis_index("c")
        sid = lax.axis_index("s")
        pltpu.sync_copy(zeros.at[pl.ds(sid * rz, rz)], acc.at[pl.ds(sid * rz, rz)])
        pltpu.sync_copy(rowi.at[sid], row_v)
        pltpu.sync_copy(coli.at[sid], col_v)
        plsc.subcore_barrier()

        def grp(g, carry):
            for b in range(u):
                j = g * u + b
                _fire_half_gather(cid, ta, tb, row_v.at[j],
                                  bufs.at[pl.ds(b * CK, CK)], gs[b])
            sd = []
            for b in range(u):
                j = g * u + b
                dst = bufs.at[pl.ds(b * CK, CK)]
                pltpu.make_async_copy(ta.at[row_v.at[j]], dst, gs[b]).wait()
                sd.append(pltpu.async_copy(dst, acc.at[col_v.at[j]],
                                           ss[b], add=True))
            for dsc in sd:
                dsc.wait()
            return carry

        lax.fori_loop(0, ng, grp, 0)
        plsc.subcore_barrier()
        pltpu.sync_copy(acc.at[pl.ds(sid * rz, rz)],
                        out.at[pl.ds(cid * rows + sid * rz, rz)])

    return k


def _make_pool3(u=4, ub=3):
    rzn = N3PAD // NS
    rzb = BPAD // NS
    nga = 44 // u
    ngb = 6 // ub

    @functools.partial(
        pl.kernel,
        out_type=(_SDS((2 * N3PAD, 32), F32), _SDS((N3PAD, 16), F32),
                  _SDS((2 * BPAD, 32), F32), _SDS((BPAD, 16), F32)),
        mesh=_MESH,
        compiler_params=_SC_PARAMS,
        scratch_types=[
            pltpu.VMEM((44, CK), jnp.int32),
            pltpu.VMEM((44, CK), jnp.int32),
            pltpu.VMEM((6, CK), jnp.int32),
            pltpu.VMEM((u * CK, 32), F32),
            pltpu.VMEM((CK, 16), F32),
            pltpu.VMEM_SHARED((N3PAD, 32), F32),
            pltpu.VMEM_SHARED((N3PAD, 16), F32),
            pltpu.VMEM_SHARED((BPAD, 32), F32),
            pltpu.VMEM_SHARED((BPAD, 16), F32),
        ] + [pltpu.SemaphoreType.DMA] * (3 * u),
    )
    def k(ta, tb, rowi, coli, bati, z32n, z16n, z32b, z16b, ones,
          outs, outc, outb, outbc,
          row_v, col_v, bat_v, bufs, ones_v, accs, accc, accb, accbc, *sems):
        gs, ss, cs = sems[:u], sems[u:2 * u], sems[2 * u:]
        cid = lax.axis_index("c")
        sid = lax.axis_index("s")
        pltpu.sync_copy(z32n.at[pl.ds(sid * rzn, rzn)],
                        accs.at[pl.ds(sid * rzn, rzn)])
        pltpu.sync_copy(z16n.at[pl.ds(sid * rzn, rzn)],
                        accc.at[pl.ds(sid * rzn, rzn)])
        pltpu.sync_copy(z32b.at[pl.ds(sid * rzb, rzb)],
                        accb.at[pl.ds(sid * rzb, rzb)])
        pltpu.sync_copy(z16b.at[pl.ds(sid * rzb, rzb)],
                        accbc.at[pl.ds(sid * rzb, rzb)])
        pltpu.sync_copy(rowi.at[sid], row_v)
        pltpu.sync_copy(coli.at[sid], col_v)
        pltpu.sync_copy(bati.at[sid], bat_v)
        pltpu.sync_copy(ones, ones_v)
        plsc.subcore_barrier()

        def grp_a(g, carry):
            for b in range(u):
                j = g * u + b
                _fire_half_gather(cid, ta, tb, row_v.at[j],
                                  bufs.at[pl.ds(b * CK, CK)], gs[b])
            sd = []
            for b in range(u):
                j = g * u + b
                dst = bufs.at[pl.ds(b * CK, CK)]
                pltpu.make_async_copy(ta.at[row_v.at[j]], dst, gs[b]).wait()
                sd.append(pltpu.async_copy(dst, accs.at[col_v.at[j]],
                                           ss[b], add=True))

                @pl.when(cid == 0)
                def _(j=j, b=b):
                    pltpu.async_copy(ones_v, accc.at[col_v.at[j]],
                                     cs[b], add=True)

            for b in range(u):
                sd[b].wait()
                j = g * u + b

                @pl.when(cid == 0)
                def _(j=j, b=b):
                    pltpu.make_async_copy(ones_v, accc.at[col_v.at[j]],
                                          cs[b]).wait()

            return carry

        lax.fori_loop(0, nga, grp_a, 0)

        def grp_b(g, carry):
            for b in range(ub):
                j = g * ub + b
                _fire_half_gather(cid, ta, tb,
                                  pl.ds(sid * 768 + j * CK, CK),
                                  bufs.at[pl.ds(b * CK, CK)], gs[b])
            sd = []
            for b in range(ub):
                j = g * ub + b
                dst = bufs.at[pl.ds(b * CK, CK)]
                pltpu.make_async_copy(
                    ta.at[pl.ds(sid * 768 + j * CK, CK)], dst, gs[b]).wait()
                sd.append(pltpu.async_copy(dst, accb.at[bat_v.at[j]],
                                           ss[b], add=True))

                @pl.when(cid == 0)
                def _(j=j, b=b):
                    pltpu.async_copy(ones_v, accbc.at[bat_v.at[j]],
                                     cs[b], add=True)

            for b in range(ub):
                sd[b].wait()
                j = g * ub + b

                @pl.when(cid == 0)
                def _(j=j, b=b):
                    pltpu.make_async_copy(ones_v, accbc.at[bat_v.at[j]],
                                          cs[b]).wait()

            return carry

        lax.fori_loop(0, ngb, grp_b, 0)
        plsc.subcore_barrier()
        pltpu.sync_copy(accs.at[pl.ds(sid * rzn, rzn)],
                        outs.at[pl.ds(cid * N3PAD + sid * rzn, rzn)])
        pltpu.sync_copy(accb.at[pl.ds(sid * rzb, rzb)],
                        outb.at[pl.ds(cid * BPAD + sid * rzb, rzb)])

        @pl.when(cid == 0)
        def _():
            pltpu.sync_copy(accc.at[pl.ds(sid * rzn, rzn)],
                            outc.at[pl.ds(sid * rzn, rzn)])
            pltpu.sync_copy(accbc.at[pl.ds(sid * rzb, rzb)],
                            outbc.at[pl.ds(sid * rzb, rzb)])

    return k


def _make_pool_batch(nchunks, ub=5):
    ept = nchunks * CK
    rzb = BPAD // NS
    ng = nchunks // ub

    @functools.partial(
        pl.kernel,
        out_type=(_SDS((2 * BPAD, 32), F32), _SDS((BPAD, 16), F32)),
        mesh=_MESH,
        compiler_params=_SC_PARAMS,
        scratch_types=[
            pltpu.VMEM((nchunks, CK), jnp.int32),
            pltpu.VMEM((ub * CK, 32), F32),
            pltpu.VMEM((CK, 16), F32),
            pltpu.VMEM_SHARED((BPAD, 32), F32),
            pltpu.VMEM_SHARED((BPAD, 16), F32),
        ] + [pltpu.SemaphoreType.DMA] * (3 * ub),
    )
    def k(ta, tb, coli, z32b, z16b, ones, outs, outc,
          col_v, bufs, ones_v, acc, accc, *sems):
        gs, ss, cs = sems[:ub], sems[ub:2 * ub], sems[2 * ub:]
        cid = lax.axis_index("c")
        sid = lax.axis_index("s")
        pltpu.sync_copy(z32b.at[pl.ds(sid * rzb, rzb)],
                        acc.at[pl.ds(sid * rzb, rzb)])
        pltpu.sync_copy(z16b.at[pl.ds(sid * rzb, rzb)],
                        accc.at[pl.ds(sid * rzb, rzb)])
        pltpu.sync_copy(coli.at[sid], col_v)
        pltpu.sync_copy(ones, ones_v)
        plsc.subcore_barrier()

        def grp(g, carry):
            for b in range(ub):
                j = g * ub + b
                _fire_half_gather(cid, ta, tb,
                                  pl.ds(sid * ept + j * CK, CK),
                                  bufs.at[pl.ds(b * CK, CK)], gs[b])
            sd = []
            for b in range(ub):
                j = g * ub + b
                dst = bufs.at[pl.ds(b * CK, CK)]
                pltpu.make_async_copy(
                    ta.at[pl.ds(sid * ept + j * CK, CK)], dst, gs[b]).wait()
                sd.append(pltpu.async_copy(dst, acc.at[col_v.at[j]],
                                           ss[b], add=True))

                @pl.when(cid == 0)
                def _(j=j, b=b):
                    pltpu.async_copy(ones_v, accc.at[col_v.at[j]],
                                     cs[b], add=True)

            for b in range(ub):
                sd[b].wait()
                j = g * ub + b

                @pl.when(cid == 0)
                def _(j=j, b=b):
                    pltpu.make_async_copy(ones_v, accc.at[col_v.at[j]],
                                          cs[b]).wait()

            return carry

        lax.fori_loop(0, ng, grp, 0)
        plsc.subcore_barrier()
        pltpu.sync_copy(acc.at[pl.ds(sid * rzb, rzb)],
                        outs.at[pl.ds(cid * BPAD + sid * rzb, rzb)])

        @pl.when(cid == 0)
        def _():
            pltpu.sync_copy(accc.at[pl.ds(sid * rzb, rzb)],
                            outc.at[pl.ds(sid * rzb, rzb)])

    return k


def _elu(a):
    return jnp.where(a > 0, a, jnp.exp(jnp.minimum(a, 0.0)) - 1.0)


def _make_msg(m_in_pad, m_out, be=512):
    kin = m_in_pad * 128

    def body(xg, ea, wa, ba, wb2, bb2, out):
        h = jnp.maximum(ea[...] @ wa[...] + ba[...], 0.0)
        xgv = xg[...]
        v = jnp.concatenate(
            [(xgv[:, i:i + 1] * h).astype(jnp.bfloat16)
             for i in range(m_in_pad)], axis=1)
        out[...] = (
            lax.dot_general(v, wb2[...], (((1,), (0,)), ((), ())),
                            preferred_element_type=F32)
            + xgv @ bb2[...])

    return pl.pallas_call(
        body,
        grid=(EPAD // be,),
        in_specs=[
            pl.BlockSpec((be, m_in_pad), lambda i: (i, 0)),
            pl.BlockSpec((be, 8), lambda i: (i, 0)),
            pl.BlockSpec((8, 128), lambda i: (0, 0)),
            pl.BlockSpec((1, 128), lambda i: (0, 0)),
            pl.BlockSpec((kin, m_out), lambda i: (0, 0)),
            pl.BlockSpec((m_in_pad, m_out), lambda i: (0, 0)),
        ],
        out_specs=pl.BlockSpec((be, m_out), lambda i: (i, 0)),
        out_shape=_SDS((EPAD, m_out), F32),
    )


def _make_node(m_in_pad, d, npad, split=False, bn=1024):
    nb = npad // bn

    def body(p0, p1, xb, root, bias, *outs):
        a = _elu(p0[...] + p1[...] + xb[...] @ root[...] + bias[...])
        if split:
            outs[0][...] = a[:, :32]
            outs[1][...] = a[:, 32:]
        else:
            outs[0][...] = a

    if split:
        out_specs = (pl.BlockSpec((bn, 32), lambda i: (i, 0)),
                     pl.BlockSpec((bn, 32), lambda i: (i, 0)))
        out_shape = (_SDS((npad, 32), F32), _SDS((npad, 32), F32))
    else:
        out_specs = pl.BlockSpec((bn, d), lambda i: (i, 0))
        out_shape = _SDS((npad, d), F32)

    return pl.pallas_call(
        body,
        grid=(nb,),
        in_specs=[
            pl.BlockSpec((bn, d), lambda i: (i, 0)),
            pl.BlockSpec((bn, d), lambda i: (i + nb, 0)),
            pl.BlockSpec((bn, m_in_pad), lambda i: (i, 0)),
            pl.BlockSpec((m_in_pad, d), lambda i: (0, 0)),
            pl.BlockSpec((1, d), lambda i: (0, 0)),
        ],
        out_specs=out_specs,
        out_shape=out_shape,
    )


def _make_gc_pre(bn=1024):
    nb = N3PAD // bn

    def body(s_lo, s_hi, c, iso, wrel_a, wrel_b, wroot_a, wroot_b, brel,
             ta_out, tb_out, r_out):
        cnt = jnp.maximum(c[...], 1.0)[:, 0:1]
        h3m = jnp.concatenate([s_lo[...], s_hi[...]], axis=1) / cnt
        isov = iso[...]
        t = h3m @ wrel_a[...] + isov @ wrel_b[...]
        ta_out[...] = t[:, :32]
        tb_out[...] = t[:, 32:]
        r_out[...] = h3m @ wroot_a[...] + isov @ wroot_b[...] + brel[...]

    return pl.pallas_call(
        body,
        grid=(nb,),
        in_specs=[
            pl.BlockSpec((bn, 32), lambda i: (i, 0)),
            pl.BlockSpec((bn, 32), lambda i: (i + nb, 0)),
            pl.BlockSpec((bn, 16), lambda i: (i, 0)),
            pl.BlockSpec((bn, 16), lambda i: (i, 0)),
            pl.BlockSpec((64, 64), lambda i: (0, 0)),
            pl.BlockSpec((16, 64), lambda i: (0, 0)),
            pl.BlockSpec((64, 64), lambda i: (0, 0)),
            pl.BlockSpec((16, 64), lambda i: (0, 0)),
            pl.BlockSpec((1, 64), lambda i: (0, 0)),
        ],
        out_specs=(pl.BlockSpec((bn, 32), lambda i: (i, 0)),
                   pl.BlockSpec((bn, 32), lambda i: (i, 0)),
                   pl.BlockSpec((bn, 64), lambda i: (i, 0))),
        out_shape=(_SDS((N3PAD, 32), F32), _SDS((N3PAD, 32), F32),
                   _SDS((N3PAD, 64), F32)),
    )


def _make_gc_mid(bn=1024):
    nb = N3PAD // bn

    def body(a_lo, a_hi, r6, wrel, wroot, brel, ta_out, tb_out, r_out):
        h3b = _elu(jnp.concatenate([a_lo[...], a_hi[...]], axis=1) + r6[...])
        t = h3b @ wrel[...]
        ta_out[...] = t[:, :32]
        tb_out[...] = t[:, 32:]
        r_out[...] = h3b @ wroot[...] + brel[...]

    return pl.pallas_call(
        body,
        grid=(nb,),
        in_specs=[
            pl.BlockSpec((bn, 32), lambda i: (i, 0)),
            pl.BlockSpec((bn, 32), lambda i: (i + nb, 0)),
            pl.BlockSpec((bn, 64), lambda i: (i, 0)),
            pl.BlockSpec((64, 64), lambda i: (0, 0)),
            pl.BlockSpec((64, 64), lambda i: (0, 0)),
            pl.BlockSpec((1, 64), lambda i: (0, 0)),
        ],
        out_specs=(pl.BlockSpec((bn, 32), lambda i: (i, 0)),
                   pl.BlockSpec((bn, 32), lambda i: (i, 0)),
                   pl.BlockSpec((bn, 64), lambda i: (i, 0))),
        out_shape=(_SDS((N3PAD, 32), F32), _SDS((N3PAD, 32), F32),
                   _SDS((N3PAD, 64), F32)),
    )


def _make_gc_post(bn=1024):
    nb = N3PAD // bn

    def body(a_lo, a_hi, r7, fa_out, fb_out):
        a = _elu(jnp.concatenate([a_lo[...], a_hi[...]], axis=1) + r7[...])
        fa_out[...] = a[:, :32]
        fb_out[...] = a[:, 32:]

    return pl.pallas_call(
        body,
        grid=(nb,),
        in_specs=[
            pl.BlockSpec((bn, 32), lambda i: (i, 0)),
            pl.BlockSpec((bn, 32), lambda i: (i + nb, 0)),
            pl.BlockSpec((bn, 64), lambda i: (i, 0)),
        ],
        out_specs=(pl.BlockSpec((bn, 32), lambda i: (i, 0)),
                   pl.BlockSpec((bn, 32), lambda i: (i, 0))),
        out_shape=(_SDS((N3PAD, 32), F32), _SDS((N3PAD, 32), F32)),
    )


def _make_head():
    def body(s10, s11, c1, s30, s31, c3,
             w1a, w1b, b1, w2, b2, w3r, b3, out):
        cnt1 = jnp.maximum(c1[...], 1.0)[:, 0:1]
        x1 = jnp.concatenate([s10[...], s11[...]], axis=1) / cnt1
        cnt3 = jnp.maximum(c3[...], 1.0)[:, 0:1]
        x3 = jnp.concatenate([s30[...], s31[...]], axis=1) / cnt3
        y = _elu(x1 @ w1a[...] + x3 @ w1b[...] + b1[...])
        y = _elu(y @ w2[...] + b2[...])
        out[...] = jnp.sum(y * w3r[...], axis=1, keepdims=True) + b3[...]

    bs = lambda shape: pl.BlockSpec(shape, lambda i: (0, 0))
    return pl.pallas_call(
        body,
        grid=(1,),
        in_specs=[
            pl.BlockSpec((BPAD, 32), lambda i: (0, 0)),
            pl.BlockSpec((BPAD, 32), lambda i: (1, 0)),
            bs((BPAD, 16)),
            pl.BlockSpec((BPAD, 32), lambda i: (0, 0)),
            pl.BlockSpec((BPAD, 32), lambda i: (1, 0)),
            bs((BPAD, 16)),
            bs((64, 64)), bs((64, 64)), bs((1, 64)),
            bs((64, 32)), bs((1, 32)), bs((1, 32)), bs((1, 1)),
        ],
        out_specs=pl.BlockSpec((BPAD, 1), lambda i: (0, 0)),
        out_shape=_SDS((BPAD, 1), F32),
    )


def _pad_idx(idx, n_pad, fill):
    v = jnp.full((n_pad,), fill, jnp.int32)
    return v.at[: idx.shape[0]].set(idx.astype(jnp.int32))


def _prep_nnconv(Wa, ba, Wb, bb, root, bias, m_in, m_in_pad, m_out):
    wa8 = jnp.zeros((8, 128), F32).at[:6].set(Wa)
    ba2 = ba.reshape(1, 128)
    wb3 = Wb.reshape(128, m_in, m_out).transpose(1, 0, 2)
    wb2 = jnp.zeros((m_in_pad, 128, m_out), F32).at[:m_in].set(wb3)
    wb2 = wb2.reshape(m_in_pad * 128, m_out).astype(jnp.bfloat16)
    bb2 = jnp.zeros((m_in_pad, m_out), F32).at[:m_in].set(bb.reshape(m_in, m_out))
    rootp = jnp.zeros((m_in_pad, m_out), F32).at[:m_in].set(root)
    bias2 = bias.reshape(1, m_out)
    return wa8, ba2, wb2, bb2, rootp, bias2


def kernel(x, edge_index, edge_attr, batch, assignment_index_3, iso_type_3,
           edge_index_3, batch_3, W1a, b1a, W1b, b1b, root1, bias1,
           W2a, b2a, W2b, b2b, root2, bias2, W3a, b3a, W3b, b3b, root3, bias3,
           Wrel6, brel6, Wroot6, Wrel7, brel7, Wroot7,
           fc1_W, fc1_b, fc2_W, fc2_b, fc3_W, fc3_b):
    xpad = jnp.zeros((NPAD, 16), F32).at[:_N, :_F_IN].set(x)
    eapad = jnp.zeros((EPAD, 8), F32).at[:_E, :6].set(edge_attr)
    src_i = _pad_idx(edge_index[0], EPAD, 0).reshape(NW, 6, CK)
    dst_i = _pad_idx(edge_index[1], EPAD, DUM_N).reshape(NW, 6, CK)
    row3_i = _pad_idx(assignment_index_3[0], APAD, 0).reshape(NS, 44, CK)
    col3_i = _pad_idx(assignment_index_3[1], APAD, DUM_N3).reshape(NS, 44, CK)
    src3_i = _pad_idx(edge_index_3[0], E3PAD, 0).reshape(NS, 60, CK)
    dst3_i = _pad_idx(edge_index_3[1], E3PAD, DUM_N3).reshape(NS, 60, CK)
    batch_i = _pad_idx(batch, NPAD, DUM_B).reshape(NS, 6, CK)
    batch3_i = _pad_idx(batch_3, N3PAD, DUM_B).reshape(NS, 15, CK)
    isopad = jnp.zeros((N3PAD, 16), F32).at[:_N3].set(iso_type_3)

    zN32 = jnp.zeros((NPAD, 32), F32)
    zN64 = jnp.zeros((NPAD, 64), F32)
    z32N3 = jnp.zeros((N3PAD, 32), F32)
    z16N3 = jnp.zeros((N3PAD, 16), F32)
    z32B = jnp.zeros((BPAD, 32), F32)
    z16B = jnp.zeros((BPAD, 16), F32)
    ones128 = jnp.ones((CK, 16), F32)

    p1w = _prep_nnconv(W1a, b1a, W1b, b1b, root1, bias1, _F_IN, 16, 32)
    p2w = _prep_nnconv(W2a, b2a, W2b, b2b, root2, bias2, 32, 32, 64)
    p3w = _prep_nnconv(W3a, b3a, W3b, b3b, root3, bias3, 64, 64, 64)

    xg1 = _make_gather(16, 6)(xpad, src_i)
    msg1 = _make_msg(16, 32)(xg1, eapad, *p1w[:4])
    agg1 = _make_scatter(32, 6, NPAD)(msg1, dst_i, zN32)
    h1 = _make_node(16, 32, NPAD)(agg1, agg1, xpad, p1w[4], p1w[5])

    xg2 = _make_gather(32, 6)(h1, src_i)
    msg2 = _make_msg(32, 64)(xg2, eapad, *p2w[:4])
    agg2 = _make_scatter(64, 6, NPAD)(msg2, dst_i, zN64)
    h2 = _make_node(32, 64, NPAD)(agg2, agg2, h1, p2w[4], p2w[5])

    xg3 = _make_gather(64, 6)(h2, src_i)
    msg3 = _make_msg(64, 64)(xg3, eapad, *p3w[:4])
    agg3 = _make_scatter(64, 6, NPAD)(msg3, dst_i, zN64)
    ha, hb = _make_node(64, 64, NPAD, split=True)(agg3, agg3, h2, p3w[4], p3w[5])

    s3sum, ccol, s1p, c1p = _make_pool3()(
        ha, hb, row3_i, col3_i, batch_i, z32N3, z16N3, z32B, z16B, ones128)

    t6a, t6b, r6 = _make_gc_pre()(
        s3sum, s3sum, ccol, isopad,
        Wrel6[:64], Wrel6[64:], Wroot6[:64], Wroot6[64:], brel6.reshape(1, 64))
    agg6 = _make_pool_split(60, N3PAD)(t6a, t6b, src3_i, dst3_i, z32N3)
    t7a, t7b, r7 = _make_gc_mid()(
        agg6, agg6, r6, Wrel7, Wroot7, brel7.reshape(1, 64))
    agg7 = _make_pool_split(60, N3PAD)(t7a, t7b, src3_i, dst3_i, z32N3)
    fa, fb = _make_gc_post()(agg7, agg7, r7)
    s3p, c3p = _make_pool_batch(15)(fa, fb, batch3_i, z32B, z16B, ones128)

    out = _make_head()(
        s1p, s1p, c1p, s3p, s3p, c3p,
        fc1_W[:64], fc1_W[64:], fc1_b.reshape(1, 64),
        fc2_W, fc2_b.reshape(1, 32),
        fc3_W.reshape(1, 32), fc3_b.reshape(1, 1))
    return out[:_B, 0]
